# Initial kernel scaffold; baseline (speedup 1.0000x reference)
import jax
import jax.numpy as jnp
from jax.experimental import pallas as pl


def kernel(x, edge_index, edge_attr, conditions, batch, params):
    raise NotImplementedError("write your pallas kernel here")


# speedup vs baseline: 2.6270x; 2.6270x over previous
import functools

import jax
import jax.numpy as jnp
from jax import lax
from jax.experimental import pallas as pl
from jax.experimental.pallas import tpu as pltpu
from jax.experimental.pallas import tpu_sc as plsc

F32 = jnp.float32
NP = 10240
NODE_BLK = 640
EDGE_BLK = 512
SC_K = 80


def _mlp2_body(x_ref, w1_ref, b1_ref, w2_ref, b2_ref, o_ref):
    h = jnp.dot(x_ref[...], w1_ref[...], preferred_element_type=F32) + b1_ref[...]
    h = jnp.maximum(h, 0.0)
    o_ref[...] = jnp.dot(h, w2_ref[...], preferred_element_type=F32) + b2_ref[...]


def _mlp2(x, w1, b1, w2, b2, block_rows):
    r, din = x.shape
    dh = w1.shape[1]
    dout = w2.shape[1]
    return pl.pallas_call(
        _mlp2_body,
        grid=(r // block_rows,),
        in_specs=[
            pl.BlockSpec((block_rows, din), lambda i: (i, 0)),
            pl.BlockSpec((din, dh), lambda i: (0, 0)),
            pl.BlockSpec((1, dh), lambda i: (0, 0)),
            pl.BlockSpec((dh, dout), lambda i: (0, 0)),
            pl.BlockSpec((1, dout), lambda i: (0, 0)),
        ],
        out_specs=pl.BlockSpec((block_rows, dout), lambda i: (i, 0)),
        out_shape=jax.ShapeDtypeStruct((r, dout), F32),
    )(x, w1, b1.reshape(1, -1), w2, b2.reshape(1, -1))


def _matmul_bias_body(x_ref, w_ref, b_ref, o_ref):
    o_ref[...] = jnp.dot(x_ref[...], w_ref[...], preferred_element_type=F32) + b_ref[...]


def _matmul_bias(x, w, b, block_rows):
    r, din = x.shape
    dout = w.shape[1]
    return pl.pallas_call(
        _matmul_bias_body,
        grid=(r // block_rows,),
        in_specs=[
            pl.BlockSpec((block_rows, din), lambda i: (i, 0)),
            pl.BlockSpec((din, dout), lambda i: (0, 0)),
            pl.BlockSpec((1, dout), lambda i: (0, 0)),
        ],
        out_specs=pl.BlockSpec((block_rows, dout), lambda i: (i, 0)),
        out_shape=jax.ShapeDtypeStruct((r, dout), F32),
    )(x, w, b.reshape(1, -1))


def _tables_body(xh_ref, oh_ref, a_ref, bm_ref, ud_ref, b1_ref, ga_ref, gb_ref):
    xh = xh_ref[...]
    ga = jnp.dot(xh, a_ref[...], preferred_element_type=F32)
    ga += jnp.dot(oh_ref[...], ud_ref[...], preferred_element_type=F32)
    ga_ref[...] = ga + b1_ref[...]
    gb_ref[...] = jnp.dot(xh, bm_ref[...], preferred_element_type=F32)


def _tables(xh, oh, a, bm, ud, b1):
    r = xh.shape[0]
    nb = oh.shape[1]
    h = a.shape[1]
    return pl.pallas_call(
        _tables_body,
        grid=(r // NODE_BLK,),
        in_specs=[
            pl.BlockSpec((NODE_BLK, h), lambda i: (i, 0)),
            pl.BlockSpec((NODE_BLK, nb), lambda i: (i, 0)),
            pl.BlockSpec((h, h), lambda i: (0, 0)),
            pl.BlockSpec((h, h), lambda i: (0, 0)),
            pl.BlockSpec((nb, h), lambda i: (0, 0)),
            pl.BlockSpec((1, h), lambda i: (0, 0)),
        ],
        out_specs=[
            pl.BlockSpec((NODE_BLK, h), lambda i: (i, 0)),
            pl.BlockSpec((NODE_BLK, h), lambda i: (i, 0)),
        ],
        out_shape=[
            jax.ShapeDtypeStruct((r, h), F32),
            jax.ShapeDtypeStruct((r, h), F32),
        ],
    )(xh, oh, a, bm, ud, b1.reshape(1, -1))


def _node_update_body(xh_ref, s0_ref, s1_ref, c0_ref, c1_ref, oh_ref,
                      w2e_ref, b2e_ref, p_ref, q_ref, ur_ref, b1n_ref,
                      w2n_ref, b2n_ref, o_ref):
    xh = xh_ref[...]
    s = s0_ref[...] + s1_ref[...]
    cnt = c0_ref[...] + c1_ref[...]
    sums = jnp.dot(s, w2e_ref[...], preferred_element_type=F32) + cnt * b2e_ref[...]
    agg = sums / jnp.maximum(cnt, 1.0)
    pre = jnp.dot(xh, p_ref[...], preferred_element_type=F32)
    pre += jnp.dot(agg, q_ref[...], preferred_element_type=F32)
    pre += jnp.dot(oh_ref[...], ur_ref[...], preferred_element_type=F32)
    hid = jnp.maximum(pre + b1n_ref[...], 0.0)
    o_ref[...] = jnp.dot(hid, w2n_ref[...], preferred_element_type=F32) + b2n_ref[...] + xh


def _node_update(xh, s0, s1, c0, c1, oh, w2e, b2e, p, q, ur, b1n, w2n, b2n):
    r, h = xh.shape
    nb = oh.shape[1]
    full = lambda d0, d1: pl.BlockSpec((d0, d1), lambda i: (0, 0))
    rows = lambda d1: pl.BlockSpec((NODE_BLK, d1), lambda i: (i, 0))
    return pl.pallas_call(
        _node_update_body,
        grid=(r // NODE_BLK,),
        in_specs=[rows(h), rows(h), rows(h), rows(1), rows(1), rows(nb),
                  full(h, h), full(1, h), full(h, h), full(h, h), full(nb, h),
                  full(1, h), full(h, h), full(1, h)],
        out_specs=rows(h),
        out_shape=jax.ShapeDtypeStruct((r, h), F32),
    )(xh, s0, s1, c0.reshape(-1, 1), c1.reshape(-1, 1), oh,
      w2e, b2e.reshape(1, -1), p, q, ur, b1n.reshape(1, -1),
      w2n, b2n.reshape(1, -1))


def _sc_edge_pass(row, col, ehc, ga, gb, zrow, *, first):
    e, h = ehc.shape
    nbkt = NP // h
    try:
        info = plsc.get_sparse_core_info()
        nc, ns = info.num_cores, info.num_subcores
    except Exception:
        nc, ns = 2, 16
    nw = nc * ns
    e_per_w = e // nw
    n_chunks = e_per_w // SC_K
    rows_per_s = NP // ns
    bkt_per_s = 8
    n_bkt_s = nbkt // bkt_per_s
    mesh = plsc.VectorSubcoreMesh(core_axis_name="c", subcore_axis_name="s",
                                  num_cores=nc, num_subcores=ns)

    out_type = []
    if first:
        out_type.append(jax.ShapeDtypeStruct((e, h), F32))
        out_type.append(jax.ShapeDtypeStruct((nc, nbkt, h), F32))
    out_type.append(jax.ShapeDtypeStruct((nc, NP, h), F32))

    scratch = [
        pltpu.VMEM((SC_K,), jnp.int32),
        pltpu.VMEM((SC_K,), jnp.int32),
        pltpu.VMEM((SC_K, h), F32),
        pltpu.VMEM((SC_K, h), F32),
        pltpu.VMEM((SC_K, h), F32),
        pltpu.VMEM_SHARED((NP, h), F32),
    ]
    if first:
        scratch.append(pltpu.VMEM((SC_K,), jnp.int32))
        scratch.append(pltpu.VMEM((SC_K, h), F32))
        scratch.append(pltpu.VMEM_SHARED((nbkt, h), F32))

    @functools.partial(pl.kernel, mesh=mesh, out_type=tuple(out_type),
                       scratch_types=scratch)
    def k(row_hbm, col_hbm, ehc_hbm, ga_hbm, gb_hbm, z_hbm, *rest):
        rest = list(rest)
        h_hbm = rest.pop(0) if first else None
        cnt_hbm = rest.pop(0) if first else None
        s_hbm = rest.pop(0)
        idxr_v = rest.pop(0)
        idxc_v = rest.pop(0)
        ga_v = rest.pop(0)
        gb_v = rest.pop(0)
        eh_v = rest.pop(0)
        s_sh = rest.pop(0)
        idxb_v = rest.pop(0) if first else None
        oh_v = rest.pop(0) if first else None
        cnt_sh = rest.pop(0) if first else None

        cid = lax.axis_index("c")
        sid = lax.axis_index("s")
        wid = cid * ns + sid

        nslc = pl.ds(sid * rows_per_s, rows_per_s)
        pltpu.sync_copy(z_hbm.at[nslc], s_sh.at[nslc])
        bslc = pl.ds(jnp.minimum(sid, n_bkt_s - 1) * bkt_per_s, bkt_per_s)
        if first:
            @pl.when(sid < n_bkt_s)
            def _():
                pltpu.sync_copy(z_hbm.at[bslc], cnt_sh.at[bslc])
        plsc.subcore_barrier()

        ebase = wid * e_per_w
        iota16 = lax.iota(jnp.int32, 16)

        def chunk_body(c, carry):
            base = ebase + c * SC_K
            eslc = pl.ds(base, SC_K)
            pltpu.sync_copy(row_hbm.at[eslc], idxr_v)
            pltpu.sync_copy(col_hbm.at[eslc], idxc_v)
            pltpu.sync_copy(ehc_hbm.at[eslc], eh_v)
            pltpu.sync_copy(ga_hbm.at[idxr_v], ga_v)
            pltpu.sync_copy(gb_hbm.at[idxc_v], gb_v)

            if first:
                def bkt_body(g, carry2):
                    sl = pl.ds(g * 16, 16)
                    idxb_v[sl] = lax.shift_right_logical(idxr_v[sl], 7)
                    return carry2

                lax.fori_loop(0, SC_K // 16, bkt_body, 0)

            def grp_body(g, carry2):
                lanes = (lax.rem(idxr_v[pl.ds(g * 16, 16)], jnp.int32(h))
                         if first else None)
                for ei in range(16):
                    i = g * 16 + ei
                    for j in range(h // 16):
                        sl = pl.ds(j * 16, 16)
                        v = ga_v[i, sl] + gb_v[i, sl] + eh_v[i, sl]
                        eh_v[i, sl] = jnp.maximum(v, 0.0)
                        if first:
                            oh_v[i, sl] = jnp.where(
                                iota16 + (j * 16) == lanes[ei],
                                1.0, 0.0).astype(F32)
                return carry2

            lax.fori_loop(0, SC_K // 16, grp_body, 0)
            if first:
                pltpu.sync_copy(eh_v, h_hbm.at[eslc])
                pltpu.sync_copy(oh_v, cnt_sh.at[idxb_v], add=True)
            pltpu.sync_copy(eh_v, s_sh.at[idxr_v], add=True)
            return carry

        lax.fori_loop(0, n_chunks, chunk_body, 0)
        plsc.subcore_barrier()
        pltpu.sync_copy(s_sh.at[nslc], s_hbm.at[cid, nslc])
        if first:
            @pl.when(sid < n_bkt_s)
            def _():
                pltpu.sync_copy(cnt_sh.at[bslc], cnt_hbm.at[cid, bslc])

    return k(row, col, ehc, ga, gb, zrow)


def kernel(x, edge_index, edge_attr, conditions, batch, params):
    n = x.shape[0]
    h = params['node_enc']['w2'].shape[1]
    nb = conditions.shape[0]

    row = edge_index[0].astype(jnp.int32)
    col = edge_index[1].astype(jnp.int32)
    batch = batch.astype(jnp.int32)

    xp = jnp.zeros((NP, x.shape[1]), F32).at[:n].set(x)
    oh = jnp.zeros((NP, nb), F32).at[:n].set(
        (batch[:, None] == jnp.arange(nb, dtype=jnp.int32)[None, :]).astype(F32))
    zrow = jnp.zeros((NP, h), F32)

    pn = params['node_enc']
    xh = _mlp2(xp, pn['w1'], pn['b1'], pn['w2'], pn['b2'], NODE_BLK)
    pc = params['cond_enc']
    u = _mlp2(conditions, pc['w1'], pc['b1'], pc['w2'], pc['b2'], nb)

    l0 = params['layers'][0]['edge']
    pe = params['edge_enc']
    c0m = jnp.split(l0['w1'], 4, axis=0)[2]
    ehc = _mlp2(edge_attr, pe['w1'], pe['b1'], pe['w2'] @ c0m, pe['b2'] @ c0m,
                EDGE_BLK)

    cnt0 = cnt1 = None
    h_prev = None
    for li in range(len(params['layers'])):
        lp = params['layers'][li]
        am, bm, cm, dm = jnp.split(lp['edge']['w1'], 4, axis=0)
        first = li == 0
        if not first:
            prev = params['layers'][li - 1]['edge']
            ehc = _matmul_bias(h_prev, prev['w2'] @ cm, prev['b2'] @ cm,
                               EDGE_BLK)
        ga, gb = _tables(xh, oh, am, bm, u @ dm, lp['edge']['b1'])
        outs = _sc_edge_pass(row, col, ehc, ga, gb, zrow, first=first)
        if first:
            h_prev, cnt_p, s_p = outs
            cnt0 = cnt_p[0].reshape(NP)
            cnt1 = cnt_p[1].reshape(NP)
        else:
            (s_p,) = outs
        pm, qm, rm = jnp.split(lp['node']['w1'], 3, axis=0)
        xh = _node_update(xh, s_p[0], s_p[1], cnt0, cnt1, oh,
                          lp['edge']['w2'], lp['edge']['b2'],
                          pm, qm, u @ rm, lp['node']['b1'],
                          lp['node']['w2'], lp['node']['b2'])

    pd = params['decoder']
    out = _mlp2(xh, pd['w1'], pd['b1'], pd['w2'], pd['b2'], NODE_BLK)
    return out[:n]


# speedup vs baseline: 3.2350x; 1.2314x over previous
import functools

import jax
import jax.numpy as jnp
from jax import lax
from jax.experimental import pallas as pl
from jax.experimental.pallas import tpu as pltpu
from jax.experimental.pallas import tpu_sc as plsc

F32 = jnp.float32
NP = 10240
NODE_BLK = 640
EDGE_BLK = 512
SC_K = 80


def _mlp2_body(x_ref, w1_ref, b1_ref, w2_ref, b2_ref, o_ref):
    h = jnp.dot(x_ref[...], w1_ref[...], preferred_element_type=F32) + b1_ref[...]
    h = jnp.maximum(h, 0.0)
    o_ref[...] = jnp.dot(h, w2_ref[...], preferred_element_type=F32) + b2_ref[...]


def _mlp2(x, w1, b1, w2, b2, block_rows):
    r, din = x.shape
    dh = w1.shape[1]
    dout = w2.shape[1]
    return pl.pallas_call(
        _mlp2_body,
        grid=(r // block_rows,),
        in_specs=[
            pl.BlockSpec((block_rows, din), lambda i: (i, 0)),
            pl.BlockSpec((din, dh), lambda i: (0, 0)),
            pl.BlockSpec((1, dh), lambda i: (0, 0)),
            pl.BlockSpec((dh, dout), lambda i: (0, 0)),
            pl.BlockSpec((1, dout), lambda i: (0, 0)),
        ],
        out_specs=pl.BlockSpec((block_rows, dout), lambda i: (i, 0)),
        out_shape=jax.ShapeDtypeStruct((r, dout), F32),
    )(x, w1, b1.reshape(1, -1), w2, b2.reshape(1, -1))


def _matmul_bias_body(x_ref, w_ref, b_ref, o_ref):
    o_ref[...] = jnp.dot(x_ref[...], w_ref[...], preferred_element_type=F32) + b_ref[...]


def _matmul_bias(x, w, b, block_rows):
    r, din = x.shape
    dout = w.shape[1]
    return pl.pallas_call(
        _matmul_bias_body,
        grid=(r // block_rows,),
        in_specs=[
            pl.BlockSpec((block_rows, din), lambda i: (i, 0)),
            pl.BlockSpec((din, dout), lambda i: (0, 0)),
            pl.BlockSpec((1, dout), lambda i: (0, 0)),
        ],
        out_specs=pl.BlockSpec((block_rows, dout), lambda i: (i, 0)),
        out_shape=jax.ShapeDtypeStruct((r, dout), F32),
    )(x, w, b.reshape(1, -1))


def _tables_body(xh_ref, oh_ref, a_ref, bm_ref, ud_ref, b1_ref, ga_ref, gb_ref):
    xh = xh_ref[...]
    ga = jnp.dot(xh, a_ref[...], preferred_element_type=F32)
    ga += jnp.dot(oh_ref[...], ud_ref[...], preferred_element_type=F32)
    ga_ref[...] = ga + b1_ref[...]
    gb_ref[...] = jnp.dot(xh, bm_ref[...], preferred_element_type=F32)


def _tables(xh, oh, a, bm, ud, b1):
    r = xh.shape[0]
    nb = oh.shape[1]
    h = a.shape[1]
    return pl.pallas_call(
        _tables_body,
        grid=(r // NODE_BLK,),
        in_specs=[
            pl.BlockSpec((NODE_BLK, h), lambda i: (i, 0)),
            pl.BlockSpec((NODE_BLK, nb), lambda i: (i, 0)),
            pl.BlockSpec((h, h), lambda i: (0, 0)),
            pl.BlockSpec((h, h), lambda i: (0, 0)),
            pl.BlockSpec((nb, h), lambda i: (0, 0)),
            pl.BlockSpec((1, h), lambda i: (0, 0)),
        ],
        out_specs=[
            pl.BlockSpec((NODE_BLK, h), lambda i: (i, 0)),
            pl.BlockSpec((NODE_BLK, h), lambda i: (i, 0)),
        ],
        out_shape=[
            jax.ShapeDtypeStruct((r, h), F32),
            jax.ShapeDtypeStruct((r, h), F32),
        ],
    )(xh, oh, a, bm, ud, b1.reshape(1, -1))


def _node_update_body(xh_ref, s0_ref, s1_ref, c0_ref, c1_ref, oh_ref,
                      w2e_ref, b2e_ref, p_ref, q_ref, ur_ref, b1n_ref,
                      w2n_ref, b2n_ref, o_ref):
    xh = xh_ref[...]
    s = s0_ref[...] + s1_ref[...]
    cnt = c0_ref[...] + c1_ref[...]
    sums = jnp.dot(s, w2e_ref[...], preferred_element_type=F32) + cnt * b2e_ref[...]
    agg = sums / jnp.maximum(cnt, 1.0)
    pre = jnp.dot(xh, p_ref[...], preferred_element_type=F32)
    pre += jnp.dot(agg, q_ref[...], preferred_element_type=F32)
    pre += jnp.dot(oh_ref[...], ur_ref[...], preferred_element_type=F32)
    hid = jnp.maximum(pre + b1n_ref[...], 0.0)
    o_ref[...] = jnp.dot(hid, w2n_ref[...], preferred_element_type=F32) + b2n_ref[...] + xh


def _node_update(xh, s0, s1, c0, c1, oh, w2e, b2e, p, q, ur, b1n, w2n, b2n):
    r, h = xh.shape
    nb = oh.shape[1]
    full = lambda d0, d1: pl.BlockSpec((d0, d1), lambda i: (0, 0))
    rows = lambda d1: pl.BlockSpec((NODE_BLK, d1), lambda i: (i, 0))
    return pl.pallas_call(
        _node_update_body,
        grid=(r // NODE_BLK,),
        in_specs=[rows(h), rows(h), rows(h), rows(1), rows(1), rows(nb),
                  full(h, h), full(1, h), full(h, h), full(h, h), full(nb, h),
                  full(1, h), full(h, h), full(1, h)],
        out_specs=rows(h),
        out_shape=jax.ShapeDtypeStruct((r, h), F32),
    )(xh, s0, s1, c0.reshape(-1, 1), c1.reshape(-1, 1), oh,
      w2e, b2e.reshape(1, -1), p, q, ur, b1n.reshape(1, -1),
      w2n, b2n.reshape(1, -1))


def _sc_edge_pass(row, col, ehc, ga, gb, zrow, *, first):
    e, h = ehc.shape
    nbkt = NP // h
    try:
        info = plsc.get_sparse_core_info()
        nc, ns = info.num_cores, info.num_subcores
    except Exception:
        nc, ns = 2, 16
    nw = nc * ns
    e_per_w = e // nw
    n_chunks = e_per_w // SC_K
    rows_per_s = NP // ns
    bkt_per_s = 8
    n_bkt_s = nbkt // bkt_per_s
    mesh = plsc.VectorSubcoreMesh(core_axis_name="c", subcore_axis_name="s",
                                  num_cores=nc, num_subcores=ns)

    out_type = []
    if first:
        out_type.append(jax.ShapeDtypeStruct((e, h), F32))
        out_type.append(jax.ShapeDtypeStruct((nc, nbkt, h), F32))
    out_type.append(jax.ShapeDtypeStruct((nc, NP, h), F32))

    scratch = [
        pltpu.VMEM((SC_K,), jnp.int32),
        pltpu.VMEM((SC_K,), jnp.int32),
        pltpu.VMEM((SC_K, h), F32),
        pltpu.VMEM((SC_K, h), F32),
        pltpu.VMEM((SC_K, h), F32),
        pltpu.VMEM_SHARED((NP, h), F32),
    ]
    if first:
        scratch.append(pltpu.VMEM((SC_K,), jnp.int32))
        scratch.append(pltpu.VMEM((SC_K, h), F32))
        scratch.append(pltpu.VMEM_SHARED((nbkt, h), F32))
    scratch += [pltpu.SemaphoreType.DMA] * 6

    @functools.partial(pl.kernel, mesh=mesh, out_type=tuple(out_type),
                       scratch_types=scratch)
    def k(row_hbm, col_hbm, ehc_hbm, ga_hbm, gb_hbm, z_hbm, *rest):
        rest = list(rest)
        h_hbm = rest.pop(0) if first else None
        cnt_hbm = rest.pop(0) if first else None
        s_hbm = rest.pop(0)
        idxr_v = rest.pop(0)
        idxc_v = rest.pop(0)
        ga_v = rest.pop(0)
        gb_v = rest.pop(0)
        eh_v = rest.pop(0)
        s_sh = rest.pop(0)
        idxb_v = rest.pop(0) if first else None
        oh_v = rest.pop(0) if first else None
        cnt_sh = rest.pop(0) if first else None
        sem_i, sem_i2, sem_e, sem_g, sem_g2, sem_h = [rest.pop(0) for _ in range(6)]

        cid = lax.axis_index("c")
        sid = lax.axis_index("s")
        wid = cid * ns + sid

        nslc = pl.ds(sid * rows_per_s, rows_per_s)
        pltpu.sync_copy(z_hbm.at[nslc], s_sh.at[nslc])
        bslc = pl.ds(jnp.minimum(sid, n_bkt_s - 1) * bkt_per_s, bkt_per_s)
        if first:
            @pl.when(sid < n_bkt_s)
            def _():
                pltpu.sync_copy(z_hbm.at[bslc], cnt_sh.at[bslc])
        plsc.subcore_barrier()

        ebase = wid * e_per_w
        iota16 = lax.iota(jnp.int32, 16)

        def chunk_body(c, carry):
            base = ebase + c * SC_K
            eslc = pl.ds(base, SC_K)
            d_r = pltpu.async_copy(row_hbm.at[eslc], idxr_v, sem_i)
            d_c = pltpu.async_copy(col_hbm.at[eslc], idxc_v, sem_i2)
            d_e = pltpu.async_copy(ehc_hbm.at[eslc], eh_v, sem_e)
            d_r.wait()
            d_c.wait()
            d_ga = pltpu.async_copy(ga_hbm.at[idxr_v], ga_v, sem_g)
            d_gb = pltpu.async_copy(gb_hbm.at[idxc_v], gb_v, sem_g2)
            d_e.wait()
            d_ga.wait()
            d_gb.wait()

            if first:
                def bkt_body(g, carry2):
                    sl = pl.ds(g * 16, 16)
                    idxb_v[sl] = lax.shift_right_logical(idxr_v[sl], 7)
                    return carry2

                lax.fori_loop(0, SC_K // 16, bkt_body, 0)

            def grp_body(g, carry2):
                lanes = (lax.rem(idxr_v[pl.ds(g * 16, 16)], jnp.int32(h))
                         if first else None)
                for ei in range(16):
                    i = g * 16 + ei
                    for j in range(h // 16):
                        sl = pl.ds(j * 16, 16)
                        v = ga_v[i, sl] + gb_v[i, sl] + eh_v[i, sl]
                        eh_v[i, sl] = jnp.maximum(v, 0.0)
                        if first:
                            oh_v[i, sl] = jnp.where(
                                iota16 + (j * 16) == lanes[ei],
                                1.0, 0.0).astype(F32)
                return carry2

            lax.fori_loop(0, SC_K // 16, grp_body, 0)
            if first:
                d_h = pltpu.async_copy(eh_v, h_hbm.at[eslc], sem_h)
                pltpu.sync_copy(oh_v, cnt_sh.at[idxb_v], add=True)
            pltpu.sync_copy(eh_v, s_sh.at[idxr_v], add=True)
            if first:
                d_h.wait()
            return carry

        lax.fori_loop(0, n_chunks, chunk_body, 0)
        plsc.subcore_barrier()
        pltpu.sync_copy(s_sh.at[nslc], s_hbm.at[cid, nslc])
        if first:
            @pl.when(sid < n_bkt_s)
            def _():
                pltpu.sync_copy(cnt_sh.at[bslc], cnt_hbm.at[cid, bslc])

    return k(row, col, ehc, ga, gb, zrow)


def kernel(x, edge_index, edge_attr, conditions, batch, params):
    n = x.shape[0]
    h = params['node_enc']['w2'].shape[1]
    nb = conditions.shape[0]

    row = edge_index[0].astype(jnp.int32)
    col = edge_index[1].astype(jnp.int32)
    batch = batch.astype(jnp.int32)

    xp = jnp.zeros((NP, x.shape[1]), F32).at[:n].set(x)
    oh = jnp.zeros((NP, nb), F32).at[:n].set(
        (batch[:, None] == jnp.arange(nb, dtype=jnp.int32)[None, :]).astype(F32))
    zrow = jnp.zeros((NP, h), F32)

    pn = params['node_enc']
    xh = _mlp2(xp, pn['w1'], pn['b1'], pn['w2'], pn['b2'], NODE_BLK)
    pc = params['cond_enc']
    u = _mlp2(conditions, pc['w1'], pc['b1'], pc['w2'], pc['b2'], nb)

    l0 = params['layers'][0]['edge']
    pe = params['edge_enc']
    c0m = jnp.split(l0['w1'], 4, axis=0)[2]
    ehc = _mlp2(edge_attr, pe['w1'], pe['b1'], pe['w2'] @ c0m, pe['b2'] @ c0m,
                EDGE_BLK)

    cnt0 = cnt1 = None
    h_prev = None
    for li in range(len(params['layers'])):
        lp = params['layers'][li]
        am, bm, cm, dm = jnp.split(lp['edge']['w1'], 4, axis=0)
        first = li == 0
        if not first:
            prev = params['layers'][li - 1]['edge']
            ehc = _matmul_bias(h_prev, prev['w2'] @ cm, prev['b2'] @ cm,
                               EDGE_BLK)
        ga, gb = _tables(xh, oh, am, bm, u @ dm, lp['edge']['b1'])
        outs = _sc_edge_pass(row, col, ehc, ga, gb, zrow, first=first)
        if first:
            h_prev, cnt_p, s_p = outs
            cnt0 = cnt_p[0].reshape(NP)
            cnt1 = cnt_p[1].reshape(NP)
        else:
            (s_p,) = outs
        pm, qm, rm = jnp.split(lp['node']['w1'], 3, axis=0)
        xh = _node_update(xh, s_p[0], s_p[1], cnt0, cnt1, oh,
                          lp['edge']['w2'], lp['edge']['b2'],
                          pm, qm, u @ rm, lp['node']['b1'],
                          lp['node']['w2'], lp['node']['b2'])

    pd = params['decoder']
    out = _mlp2(xh, pd['w1'], pd['b1'], pd['w2'], pd['b2'], NODE_BLK)
    return out[:n]


# speedup vs baseline: 3.2455x; 1.0033x over previous
import functools

import jax
import jax.numpy as jnp
from jax import lax
from jax.experimental import pallas as pl
from jax.experimental.pallas import tpu as pltpu
from jax.experimental.pallas import tpu_sc as plsc

F32 = jnp.float32
NP = 10240
NODE_BLK = 640
EDGE_BLK = 512
SC_K = 80


def _mlp2_body(x_ref, w1_ref, b1_ref, w2_ref, b2_ref, o_ref):
    h = jnp.dot(x_ref[...], w1_ref[...], preferred_element_type=F32) + b1_ref[...]
    h = jnp.maximum(h, 0.0)
    o_ref[...] = jnp.dot(h, w2_ref[...], preferred_element_type=F32) + b2_ref[...]


def _mlp2(x, w1, b1, w2, b2, block_rows):
    r, din = x.shape
    dh = w1.shape[1]
    dout = w2.shape[1]
    return pl.pallas_call(
        _mlp2_body,
        grid=(r // block_rows,),
        in_specs=[
            pl.BlockSpec((block_rows, din), lambda i: (i, 0)),
            pl.BlockSpec((din, dh), lambda i: (0, 0)),
            pl.BlockSpec((1, dh), lambda i: (0, 0)),
            pl.BlockSpec((dh, dout), lambda i: (0, 0)),
            pl.BlockSpec((1, dout), lambda i: (0, 0)),
        ],
        out_specs=pl.BlockSpec((block_rows, dout), lambda i: (i, 0)),
        out_shape=jax.ShapeDtypeStruct((r, dout), F32),
    )(x, w1, b1.reshape(1, -1), w2, b2.reshape(1, -1))


def _matmul_bias_body(x_ref, w_ref, b_ref, o_ref):
    o_ref[...] = jnp.dot(x_ref[...], w_ref[...], preferred_element_type=F32) + b_ref[...]


def _matmul_bias(x, w, b, block_rows):
    r, din = x.shape
    dout = w.shape[1]
    return pl.pallas_call(
        _matmul_bias_body,
        grid=(r // block_rows,),
        in_specs=[
            pl.BlockSpec((block_rows, din), lambda i: (i, 0)),
            pl.BlockSpec((din, dout), lambda i: (0, 0)),
            pl.BlockSpec((1, dout), lambda i: (0, 0)),
        ],
        out_specs=pl.BlockSpec((block_rows, dout), lambda i: (i, 0)),
        out_shape=jax.ShapeDtypeStruct((r, dout), F32),
    )(x, w, b.reshape(1, -1))


def _tables_body(xh_ref, oh_ref, a_ref, bm_ref, ud_ref, b1_ref, ga_ref, gb_ref):
    xh = xh_ref[...]
    ga = jnp.dot(xh, a_ref[...], preferred_element_type=F32)
    ga += jnp.dot(oh_ref[...], ud_ref[...], preferred_element_type=F32)
    ga_ref[...] = ga + b1_ref[...]
    gb_ref[...] = jnp.dot(xh, bm_ref[...], preferred_element_type=F32)


def _tables(xh, oh, a, bm, ud, b1):
    r = xh.shape[0]
    nb = oh.shape[1]
    h = a.shape[1]
    return pl.pallas_call(
        _tables_body,
        grid=(r // NODE_BLK,),
        in_specs=[
            pl.BlockSpec((NODE_BLK, h), lambda i: (i, 0)),
            pl.BlockSpec((NODE_BLK, nb), lambda i: (i, 0)),
            pl.BlockSpec((h, h), lambda i: (0, 0)),
            pl.BlockSpec((h, h), lambda i: (0, 0)),
            pl.BlockSpec((nb, h), lambda i: (0, 0)),
            pl.BlockSpec((1, h), lambda i: (0, 0)),
        ],
        out_specs=[
            pl.BlockSpec((NODE_BLK, h), lambda i: (i, 0)),
            pl.BlockSpec((NODE_BLK, h), lambda i: (i, 0)),
        ],
        out_shape=[
            jax.ShapeDtypeStruct((r, h), F32),
            jax.ShapeDtypeStruct((r, h), F32),
        ],
    )(xh, oh, a, bm, ud, b1.reshape(1, -1))


def _node_update_body(xh_ref, s0_ref, s1_ref, c0_ref, c1_ref, oh_ref,
                      w2e_ref, b2e_ref, p_ref, q_ref, ur_ref, b1n_ref,
                      w2n_ref, b2n_ref, o_ref):
    xh = xh_ref[...]
    s = s0_ref[...] + s1_ref[...]
    cnt = c0_ref[...] + c1_ref[...]
    sums = jnp.dot(s, w2e_ref[...], preferred_element_type=F32) + cnt * b2e_ref[...]
    agg = sums / jnp.maximum(cnt, 1.0)
    pre = jnp.dot(xh, p_ref[...], preferred_element_type=F32)
    pre += jnp.dot(agg, q_ref[...], preferred_element_type=F32)
    pre += jnp.dot(oh_ref[...], ur_ref[...], preferred_element_type=F32)
    hid = jnp.maximum(pre + b1n_ref[...], 0.0)
    o_ref[...] = jnp.dot(hid, w2n_ref[...], preferred_element_type=F32) + b2n_ref[...] + xh


def _node_update(xh, s0, s1, c0, c1, oh, w2e, b2e, p, q, ur, b1n, w2n, b2n):
    r, h = xh.shape
    nb = oh.shape[1]
    full = lambda d0, d1: pl.BlockSpec((d0, d1), lambda i: (0, 0))
    rows = lambda d1: pl.BlockSpec((NODE_BLK, d1), lambda i: (i, 0))
    return pl.pallas_call(
        _node_update_body,
        grid=(r // NODE_BLK,),
        in_specs=[rows(h), rows(h), rows(h), rows(1), rows(1), rows(nb),
                  full(h, h), full(1, h), full(h, h), full(h, h), full(nb, h),
                  full(1, h), full(h, h), full(1, h)],
        out_specs=rows(h),
        out_shape=jax.ShapeDtypeStruct((r, h), F32),
    )(xh, s0, s1, c0.reshape(-1, 1), c1.reshape(-1, 1), oh,
      w2e, b2e.reshape(1, -1), p, q, ur, b1n.reshape(1, -1),
      w2n, b2n.reshape(1, -1))


def _sc_edge_pass(row, col, ehc, ga, gb, zrow, *, first):
    e, h = ehc.shape
    nbkt = NP // h
    try:
        info = plsc.get_sparse_core_info()
        nc, ns = info.num_cores, info.num_subcores
    except Exception:
        nc, ns = 2, 16
    nw = nc * ns
    e_per_w = e // nw
    n_chunks = e_per_w // SC_K
    rows_per_s = NP // ns
    bkt_per_s = 8
    n_bkt_s = nbkt // bkt_per_s
    mesh = plsc.VectorSubcoreMesh(core_axis_name="c", subcore_axis_name="s",
                                  num_cores=nc, num_subcores=ns)

    out_type = []
    if first:
        out_type.append(jax.ShapeDtypeStruct((e, h), F32))
        out_type.append(jax.ShapeDtypeStruct((nc, nbkt, h), F32))
    out_type.append(jax.ShapeDtypeStruct((nc, NP, h), F32))

    scratch = [
        pltpu.VMEM((SC_K,), jnp.int32),
        pltpu.VMEM((SC_K,), jnp.int32),
        pltpu.VMEM((SC_K, h), F32),
        pltpu.VMEM((SC_K, h), F32),
        pltpu.VMEM((SC_K, h), F32),
        pltpu.VMEM_SHARED((NP, h), F32),
    ]
    if first:
        scratch.append(pltpu.VMEM((SC_K,), jnp.int32))
        scratch.append(pltpu.VMEM((SC_K, h), F32))
        scratch.append(pltpu.VMEM_SHARED((nbkt, h), F32))
    scratch += [pltpu.SemaphoreType.DMA] * 6

    @functools.partial(pl.kernel, mesh=mesh, out_type=tuple(out_type),
                       scratch_types=scratch)
    def k(row_hbm, col_hbm, ehc_hbm, ga_hbm, gb_hbm, z_hbm, *rest):
        rest = list(rest)
        h_hbm = rest.pop(0) if first else None
        cnt_hbm = rest.pop(0) if first else None
        s_hbm = rest.pop(0)
        idxr_v = rest.pop(0)
        idxc_v = rest.pop(0)
        ga_v = rest.pop(0)
        gb_v = rest.pop(0)
        eh_v = rest.pop(0)
        s_sh = rest.pop(0)
        idxb_v = rest.pop(0) if first else None
        oh_v = rest.pop(0) if first else None
        cnt_sh = rest.pop(0) if first else None
        sem_i, sem_i2, sem_e, sem_g, sem_g2, sem_h = [rest.pop(0) for _ in range(6)]

        cid = lax.axis_index("c")
        sid = lax.axis_index("s")
        wid = cid * ns + sid

        nslc = pl.ds(sid * rows_per_s, rows_per_s)
        pltpu.sync_copy(z_hbm.at[nslc], s_sh.at[nslc])
        bslc = pl.ds(jnp.minimum(sid, n_bkt_s - 1) * bkt_per_s, bkt_per_s)
        if first:
            @pl.when(sid < n_bkt_s)
            def _():
                pltpu.sync_copy(z_hbm.at[bslc], cnt_sh.at[bslc])
        plsc.subcore_barrier()

        ebase = wid * e_per_w
        iota16 = lax.iota(jnp.int32, 16)

        def chunk_body(c, carry):
            base = ebase + c * SC_K
            eslc = pl.ds(base, SC_K)
            d_r = pltpu.async_copy(row_hbm.at[eslc], idxr_v, sem_i)
            d_c = pltpu.async_copy(col_hbm.at[eslc], idxc_v, sem_i2)
            d_e = pltpu.async_copy(ehc_hbm.at[eslc], eh_v, sem_e)
            d_r.wait()
            d_c.wait()
            d_ga = pltpu.async_copy(ga_hbm.at[idxr_v], ga_v, sem_g)
            d_gb = pltpu.async_copy(gb_hbm.at[idxc_v], gb_v, sem_g2)
            d_e.wait()
            d_ga.wait()
            d_gb.wait()

            if first:
                def bkt_body(g, carry2):
                    sl = pl.ds(g * 16, 16)
                    idxb_v[sl] = lax.shift_right_logical(idxr_v[sl], 7)
                    return carry2

                lax.fori_loop(0, SC_K // 16, bkt_body, 0)

            def grp_body(g, carry2):
                lanes = (lax.rem(idxr_v[pl.ds(g * 16, 16)], jnp.int32(h))
                         if first else None)
                for ei in range(16):
                    i = g * 16 + ei
                    for j in range(h // 16):
                        sl = pl.ds(j * 16, 16)
                        v = ga_v[i, sl] + gb_v[i, sl] + eh_v[i, sl]
                        eh_v[i, sl] = jnp.maximum(v, 0.0)
                        if first:
                            oh_v[i, sl] = jnp.where(
                                iota16 + (j * 16) == lanes[ei],
                                1.0, 0.0).astype(F32)
                return carry2

            lax.fori_loop(0, SC_K // 16, grp_body, 0)
            if first:
                d_h = pltpu.async_copy(eh_v, h_hbm.at[eslc], sem_h)
                d_oh = pltpu.async_copy(oh_v, cnt_sh.at[idxb_v], sem_i,
                                        add=True)
            d_s = pltpu.async_copy(eh_v, s_sh.at[idxr_v], sem_i2, add=True)
            if first:
                d_oh.wait()
                d_h.wait()
            d_s.wait()
            return carry

        lax.fori_loop(0, n_chunks, chunk_body, 0)
        plsc.subcore_barrier()
        pltpu.sync_copy(s_sh.at[nslc], s_hbm.at[cid, nslc])
        if first:
            @pl.when(sid < n_bkt_s)
            def _():
                pltpu.sync_copy(cnt_sh.at[bslc], cnt_hbm.at[cid, bslc])

    return k(row, col, ehc, ga, gb, zrow)


def kernel(x, edge_index, edge_attr, conditions, batch, params):
    n = x.shape[0]
    h = params['node_enc']['w2'].shape[1]
    nb = conditions.shape[0]

    row = edge_index[0].astype(jnp.int32)
    col = edge_index[1].astype(jnp.int32)
    batch = batch.astype(jnp.int32)

    xp = jnp.zeros((NP, x.shape[1]), F32).at[:n].set(x)
    oh = jnp.zeros((NP, nb), F32).at[:n].set(
        (batch[:, None] == jnp.arange(nb, dtype=jnp.int32)[None, :]).astype(F32))
    zrow = jnp.zeros((NP, h), F32)

    pn = params['node_enc']
    xh = _mlp2(xp, pn['w1'], pn['b1'], pn['w2'], pn['b2'], NODE_BLK)
    pc = params['cond_enc']
    u = _mlp2(conditions, pc['w1'], pc['b1'], pc['w2'], pc['b2'], nb)

    l0 = params['layers'][0]['edge']
    pe = params['edge_enc']
    c0m = jnp.split(l0['w1'], 4, axis=0)[2]
    ehc = _mlp2(edge_attr, pe['w1'], pe['b1'], pe['w2'] @ c0m, pe['b2'] @ c0m,
                EDGE_BLK)

    cnt0 = cnt1 = None
    h_prev = None
    for li in range(len(params['layers'])):
        lp = params['layers'][li]
        am, bm, cm, dm = jnp.split(lp['edge']['w1'], 4, axis=0)
        first = li == 0
        if not first:
            prev = params['layers'][li - 1]['edge']
            ehc = _matmul_bias(h_prev, prev['w2'] @ cm, prev['b2'] @ cm,
                               EDGE_BLK)
        ga, gb = _tables(xh, oh, am, bm, u @ dm, lp['edge']['b1'])
        outs = _sc_edge_pass(row, col, ehc, ga, gb, zrow, first=first)
        if first:
            h_prev, cnt_p, s_p = outs
            cnt0 = cnt_p[0].reshape(NP)
            cnt1 = cnt_p[1].reshape(NP)
        else:
            (s_p,) = outs
        pm, qm, rm = jnp.split(lp['node']['w1'], 3, axis=0)
        xh = _node_update(xh, s_p[0], s_p[1], cnt0, cnt1, oh,
                          lp['edge']['w2'], lp['edge']['b2'],
                          pm, qm, u @ rm, lp['node']['b1'],
                          lp['node']['w2'], lp['node']['b2'])

    pd = params['decoder']
    out = _mlp2(xh, pd['w1'], pd['b1'], pd['w2'], pd['b2'], NODE_BLK)
    return out[:n]


# speedup vs baseline: 3.2900x; 1.0137x over previous
import functools

import jax
import jax.numpy as jnp
from jax import lax
from jax.experimental import pallas as pl
from jax.experimental.pallas import tpu as pltpu
from jax.experimental.pallas import tpu_sc as plsc

F32 = jnp.float32
NP = 10240
NODE_BLK = 640
EDGE_BLK = 512
SC_K = 80


def _mlp2_body(x_ref, w1_ref, b1_ref, w2_ref, b2_ref, o_ref):
    h = jnp.dot(x_ref[...], w1_ref[...], preferred_element_type=F32) + b1_ref[...]
    h = jnp.maximum(h, 0.0)
    o_ref[...] = jnp.dot(h, w2_ref[...], preferred_element_type=F32) + b2_ref[...]


def _mlp2(x, w1, b1, w2, b2, block_rows):
    r, din = x.shape
    dh = w1.shape[1]
    dout = w2.shape[1]
    return pl.pallas_call(
        _mlp2_body,
        grid=(r // block_rows,),
        in_specs=[
            pl.BlockSpec((block_rows, din), lambda i: (i, 0)),
            pl.BlockSpec((din, dh), lambda i: (0, 0)),
            pl.BlockSpec((1, dh), lambda i: (0, 0)),
            pl.BlockSpec((dh, dout), lambda i: (0, 0)),
            pl.BlockSpec((1, dout), lambda i: (0, 0)),
        ],
        out_specs=pl.BlockSpec((block_rows, dout), lambda i: (i, 0)),
        out_shape=jax.ShapeDtypeStruct((r, dout), F32),
    )(x, w1, b1.reshape(1, -1), w2, b2.reshape(1, -1))


def _matmul_bias_body(x_ref, w_ref, b_ref, o_ref):
    o_ref[...] = jnp.dot(x_ref[...], w_ref[...], preferred_element_type=F32) + b_ref[...]


def _matmul_bias(x, w, b, block_rows):
    r, din = x.shape
    dout = w.shape[1]
    return pl.pallas_call(
        _matmul_bias_body,
        grid=(r // block_rows,),
        in_specs=[
            pl.BlockSpec((block_rows, din), lambda i: (i, 0)),
            pl.BlockSpec((din, dout), lambda i: (0, 0)),
            pl.BlockSpec((1, dout), lambda i: (0, 0)),
        ],
        out_specs=pl.BlockSpec((block_rows, dout), lambda i: (i, 0)),
        out_shape=jax.ShapeDtypeStruct((r, dout), F32),
    )(x, w, b.reshape(1, -1))


def _enc_tables_body(x_ref, oh_ref, w1_ref, b1_ref, w2_ref, b2_ref,
                     a_ref, bm_ref, ud_ref, b1e_ref,
                     xh_ref, ga_ref, gb_ref):
    hid = jnp.dot(x_ref[...], w1_ref[...], preferred_element_type=F32) + b1_ref[...]
    hid = jnp.maximum(hid, 0.0)
    xh = jnp.dot(hid, w2_ref[...], preferred_element_type=F32) + b2_ref[...]
    xh_ref[...] = xh
    ga = jnp.dot(xh, a_ref[...], preferred_element_type=F32)
    ga += jnp.dot(oh_ref[...], ud_ref[...], preferred_element_type=F32)
    ga_ref[...] = ga + b1e_ref[...]
    gb_ref[...] = jnp.dot(xh, bm_ref[...], preferred_element_type=F32)


def _enc_tables(x, oh, w1, b1, w2, b2, a, bm, ud, b1e):
    r, din = x.shape
    nb = oh.shape[1]
    h = w2.shape[1]
    full = lambda d0, d1: pl.BlockSpec((d0, d1), lambda i: (0, 0))
    rows = lambda d1: pl.BlockSpec((NODE_BLK, d1), lambda i: (i, 0))
    return pl.pallas_call(
        _enc_tables_body,
        grid=(r // NODE_BLK,),
        in_specs=[rows(din), rows(nb), full(din, h), full(1, h), full(h, h),
                  full(1, h), full(h, h), full(h, h), full(nb, h), full(1, h)],
        out_specs=[rows(h), rows(h), rows(h)],
        out_shape=[jax.ShapeDtypeStruct((r, h), F32)] * 3,
    )(x, oh, w1, b1.reshape(1, -1), w2, b2.reshape(1, -1),
      a, bm, ud, b1e.reshape(1, -1))


def _node_update_body(nxt, xh_ref, s0_ref, s1_ref, c0_ref, c1_ref, oh_ref,
                      w2e_ref, b2e_ref, p_ref, q_ref, ur_ref, b1n_ref,
                      w2n_ref, b2n_ref, *rest):
    xh = xh_ref[...]
    s = s0_ref[...] + s1_ref[...]
    cnt = c0_ref[...] + c1_ref[...]
    sums = jnp.dot(s, w2e_ref[...], preferred_element_type=F32) + cnt * b2e_ref[...]
    agg = sums / jnp.maximum(cnt, 1.0)
    pre = jnp.dot(xh, p_ref[...], preferred_element_type=F32)
    pre += jnp.dot(agg, q_ref[...], preferred_element_type=F32)
    pre += jnp.dot(oh_ref[...], ur_ref[...], preferred_element_type=F32)
    hid = jnp.maximum(pre + b1n_ref[...], 0.0)
    xh2 = jnp.dot(hid, w2n_ref[...], preferred_element_type=F32) + b2n_ref[...] + xh
    if nxt == 'tables':
        a_ref, bm_ref, ud_ref, b1e_ref, o_ref, ga_ref, gb_ref = rest
        o_ref[...] = xh2
        ga = jnp.dot(xh2, a_ref[...], preferred_element_type=F32)
        ga += jnp.dot(oh_ref[...], ud_ref[...], preferred_element_type=F32)
        ga_ref[...] = ga + b1e_ref[...]
        gb_ref[...] = jnp.dot(xh2, bm_ref[...], preferred_element_type=F32)
    else:
        w1d_ref, b1d_ref, w2d_ref, b2d_ref, o_ref = rest
        hd = jnp.dot(xh2, w1d_ref[...], preferred_element_type=F32) + b1d_ref[...]
        hd = jnp.maximum(hd, 0.0)
        o_ref[...] = jnp.dot(hd, w2d_ref[...], preferred_element_type=F32) + b2d_ref[...]


def _node_update(xh, s0, s1, c0, c1, oh, w2e, b2e, p, q, ur, b1n, w2n, b2n,
                 nxt, extra):
    r, h = xh.shape
    nb = oh.shape[1]
    full = lambda d0, d1: pl.BlockSpec((d0, d1), lambda i: (0, 0))
    rows = lambda d1: pl.BlockSpec((NODE_BLK, d1), lambda i: (i, 0))
    in_specs = [rows(h), rows(h), rows(h), rows(1), rows(1), rows(nb),
                full(h, h), full(1, h), full(h, h), full(h, h), full(nb, h),
                full(1, h), full(h, h), full(1, h)]
    args = [xh, s0, s1, c0.reshape(-1, 1), c1.reshape(-1, 1), oh,
            w2e, b2e.reshape(1, -1), p, q, ur, b1n.reshape(1, -1),
            w2n, b2n.reshape(1, -1)]
    if nxt == 'tables':
        a, bm, ud, b1e = extra
        in_specs += [full(h, h), full(h, h), full(nb, h), full(1, h)]
        args += [a, bm, ud, b1e.reshape(1, -1)]
        out_specs = [rows(h), rows(h), rows(h)]
        out_shape = [jax.ShapeDtypeStruct((r, h), F32)] * 3
    else:
        w1d, b1d, w2d, b2d = extra
        dh = w1d.shape[1]
        dout = w2d.shape[1]
        in_specs += [full(h, dh), full(1, dh), full(dh, dout), full(1, dout)]
        args += [w1d, b1d.reshape(1, -1), w2d, b2d.reshape(1, -1)]
        out_specs = rows(dout)
        out_shape = jax.ShapeDtypeStruct((r, dout), F32)
    return pl.pallas_call(
        functools.partial(_node_update_body, nxt),
        grid=(r // NODE_BLK,),
        in_specs=in_specs,
        out_specs=out_specs,
        out_shape=out_shape,
    )(*args)


def _sc_edge_pass(row, col, ehc, ga, gb, zrow, *, first):
    e, h = ehc.shape
    nbkt = NP // h
    try:
        info = plsc.get_sparse_core_info()
        nc, ns = info.num_cores, info.num_subcores
    except Exception:
        nc, ns = 2, 16
    nw = nc * ns
    e_per_w = e // nw
    n_chunks = e_per_w // SC_K
    rows_per_s = NP // ns
    bkt_per_s = 8
    n_bkt_s = nbkt // bkt_per_s
    mesh = plsc.VectorSubcoreMesh(core_axis_name="c", subcore_axis_name="s",
                                  num_cores=nc, num_subcores=ns)

    out_type = []
    if first:
        out_type.append(jax.ShapeDtypeStruct((e, h), F32))
        out_type.append(jax.ShapeDtypeStruct((nc, nbkt, h), F32))
    out_type.append(jax.ShapeDtypeStruct((nc, NP, h), F32))

    scratch = [
        pltpu.VMEM((SC_K,), jnp.int32),
        pltpu.VMEM((SC_K,), jnp.int32),
        pltpu.VMEM((SC_K, h), F32),
        pltpu.VMEM((SC_K, h), F32),
        pltpu.VMEM((SC_K, h), F32),
        pltpu.VMEM_SHARED((NP, h), F32),
    ]
    if first:
        scratch.append(pltpu.VMEM((SC_K,), jnp.int32))
        scratch.append(pltpu.VMEM((SC_K, h), F32))
        scratch.append(pltpu.VMEM_SHARED((nbkt, h), F32))
    scratch += [pltpu.SemaphoreType.DMA] * 6

    @functools.partial(pl.kernel, mesh=mesh, out_type=tuple(out_type),
                       scratch_types=scratch)
    def k(row_hbm, col_hbm, ehc_hbm, ga_hbm, gb_hbm, z_hbm, *rest):
        rest = list(rest)
        h_hbm = rest.pop(0) if first else None
        cnt_hbm = rest.pop(0) if first else None
        s_hbm = rest.pop(0)
        idxr_v = rest.pop(0)
        idxc_v = rest.pop(0)
        ga_v = rest.pop(0)
        gb_v = rest.pop(0)
        eh_v = rest.pop(0)
        s_sh = rest.pop(0)
        idxb_v = rest.pop(0) if first else None
        oh_v = rest.pop(0) if first else None
        cnt_sh = rest.pop(0) if first else None
        sem_i, sem_i2, sem_e, sem_g, sem_g2, sem_h = [rest.pop(0) for _ in range(6)]

        cid = lax.axis_index("c")
        sid = lax.axis_index("s")
        wid = cid * ns + sid

        nslc = pl.ds(sid * rows_per_s, rows_per_s)
        pltpu.sync_copy(z_hbm.at[nslc], s_sh.at[nslc])
        bslc = pl.ds(jnp.minimum(sid, n_bkt_s - 1) * bkt_per_s, bkt_per_s)
        if first:
            @pl.when(sid < n_bkt_s)
            def _():
                pltpu.sync_copy(z_hbm.at[bslc], cnt_sh.at[bslc])
        plsc.subcore_barrier()

        ebase = wid * e_per_w
        iota16 = lax.iota(jnp.int32, 16)

        def chunk_body(c, carry):
            base = ebase + c * SC_K
            eslc = pl.ds(base, SC_K)
            d_r = pltpu.async_copy(row_hbm.at[eslc], idxr_v, sem_i)
            d_c = pltpu.async_copy(col_hbm.at[eslc], idxc_v, sem_i2)
            d_e = pltpu.async_copy(ehc_hbm.at[eslc], eh_v, sem_e)
            d_r.wait()
            d_c.wait()
            d_ga = pltpu.async_copy(ga_hbm.at[idxr_v], ga_v, sem_g)
            d_gb = pltpu.async_copy(gb_hbm.at[idxc_v], gb_v, sem_g2)
            d_e.wait()
            d_ga.wait()
            d_gb.wait()

            if first:
                def bkt_body(g, carry2):
                    sl = pl.ds(g * 16, 16)
                    idxb_v[sl] = lax.shift_right_logical(idxr_v[sl], 7)
                    return carry2

                lax.fori_loop(0, SC_K // 16, bkt_body, 0)

            def grp_body(g, carry2):
                lanes = (lax.rem(idxr_v[pl.ds(g * 16, 16)], jnp.int32(h))
                         if first else None)
                for ei in range(16):
                    i = g * 16 + ei
                    for j in range(h // 16):
                        sl = pl.ds(j * 16, 16)
                        v = ga_v[i, sl] + gb_v[i, sl] + eh_v[i, sl]
                        eh_v[i, sl] = jnp.maximum(v, 0.0)
                        if first:
                            oh_v[i, sl] = jnp.where(
                                iota16 + (j * 16) == lanes[ei],
                                1.0, 0.0).astype(F32)
                return carry2

            lax.fori_loop(0, SC_K // 16, grp_body, 0)
            if first:
                d_h = pltpu.async_copy(eh_v, h_hbm.at[eslc], sem_h)
                d_oh = pltpu.async_copy(oh_v, cnt_sh.at[idxb_v], sem_i,
                                        add=True)
            d_s = pltpu.async_copy(eh_v, s_sh.at[idxr_v], sem_i2, add=True)
            if first:
                d_oh.wait()
                d_h.wait()
            d_s.wait()
            return carry

        lax.fori_loop(0, n_chunks, chunk_body, 0)
        plsc.subcore_barrier()
        pltpu.sync_copy(s_sh.at[nslc], s_hbm.at[cid, nslc])
        if first:
            @pl.when(sid < n_bkt_s)
            def _():
                pltpu.sync_copy(cnt_sh.at[bslc], cnt_hbm.at[cid, bslc])

    return k(row, col, ehc, ga, gb, zrow)


def kernel(x, edge_index, edge_attr, conditions, batch, params):
    n = x.shape[0]
    h = params['node_enc']['w2'].shape[1]
    nb = conditions.shape[0]

    row = edge_index[0].astype(jnp.int32)
    col = edge_index[1].astype(jnp.int32)
    batch = batch.astype(jnp.int32)

    xp = jnp.zeros((NP, x.shape[1]), F32).at[:n].set(x)
    oh = jnp.zeros((NP, nb), F32).at[:n].set(
        (batch[:, None] == jnp.arange(nb, dtype=jnp.int32)[None, :]).astype(F32))
    zrow = jnp.zeros((NP, h), F32)

    l0 = params['layers'][0]
    a0, b0m, c0m, d0m = jnp.split(l0['edge']['w1'], 4, axis=0)
    pc = params['cond_enc']
    u = _mlp2(conditions, pc['w1'], pc['b1'], pc['w2'], pc['b2'], nb)
    pn = params['node_enc']
    xh, ga, gb = _enc_tables(xp, oh, pn['w1'], pn['b1'], pn['w2'], pn['b2'],
                             a0, b0m, u @ d0m, l0['edge']['b1'])

    pe = params['edge_enc']
    ehc = _mlp2(edge_attr, pe['w1'], pe['b1'], pe['w2'] @ c0m, pe['b2'] @ c0m,
                EDGE_BLK)

    l1 = params['layers'][1]
    a1, b1m, c1m, d1m = jnp.split(l1['edge']['w1'], 4, axis=0)
    pd = params['decoder']

    for li in range(2):
        lp = params['layers'][li]
        first = li == 0
        if not first:
            prev = params['layers'][0]['edge']
            ehc = _matmul_bias(h_prev, prev['w2'] @ c1m, prev['b2'] @ c1m,
                               EDGE_BLK)
        outs = _sc_edge_pass(row, col, ehc, ga, gb, zrow, first=first)
        if first:
            h_prev, cnt_p, s_p = outs
            cnt0 = cnt_p[0].reshape(NP)
            cnt1 = cnt_p[1].reshape(NP)
        else:
            (s_p,) = outs
        pm, qm, rm = jnp.split(lp['node']['w1'], 3, axis=0)
        if first:
            xh, ga, gb = _node_update(
                xh, s_p[0], s_p[1], cnt0, cnt1, oh,
                lp['edge']['w2'], lp['edge']['b2'],
                pm, qm, u @ rm, lp['node']['b1'],
                lp['node']['w2'], lp['node']['b2'],
                'tables', (a1, b1m, u @ d1m, l1['edge']['b1']))
        else:
            out = _node_update(
                xh, s_p[0], s_p[1], cnt0, cnt1, oh,
                lp['edge']['w2'], lp['edge']['b2'],
                pm, qm, u @ rm, lp['node']['b1'],
                lp['node']['w2'], lp['node']['b2'],
                'decoder', (pd['w1'], pd['b1'], pd['w2'], pd['b2']))

    return out[:n]


# speedup vs baseline: 4.0504x; 1.2311x over previous
import functools

import jax
import jax.numpy as jnp
from jax import lax
from jax.experimental import pallas as pl
from jax.experimental.pallas import tpu as pltpu
from jax.experimental.pallas import tpu_sc as plsc

F32 = jnp.float32
NP = 10240
NODE_BLK = 640
EDGE_BLK = 1600
SC_K = 80


def _mlp2_body(x_ref, w1_ref, b1_ref, w2_ref, b2_ref, o_ref):
    h = jnp.dot(x_ref[...], w1_ref[...], preferred_element_type=F32) + b1_ref[...]
    h = jnp.maximum(h, 0.0)
    o_ref[...] = jnp.dot(h, w2_ref[...], preferred_element_type=F32) + b2_ref[...]


def _mlp2(x, w1, b1, w2, b2, block_rows):
    r, din = x.shape
    dh = w1.shape[1]
    dout = w2.shape[1]
    return pl.pallas_call(
        _mlp2_body,
        grid=(r // block_rows,),
        in_specs=[
            pl.BlockSpec((block_rows, din), lambda i: (i, 0)),
            pl.BlockSpec((din, dh), lambda i: (0, 0)),
            pl.BlockSpec((1, dh), lambda i: (0, 0)),
            pl.BlockSpec((dh, dout), lambda i: (0, 0)),
            pl.BlockSpec((1, dout), lambda i: (0, 0)),
        ],
        out_specs=pl.BlockSpec((block_rows, dout), lambda i: (i, 0)),
        out_shape=jax.ShapeDtypeStruct((r, dout), F32),
    )(x, w1, b1.reshape(1, -1), w2, b2.reshape(1, -1))


def _matmul_bias_body(x_ref, w_ref, b_ref, o_ref):
    o_ref[...] = jnp.dot(x_ref[...], w_ref[...], preferred_element_type=F32) + b_ref[...]


def _matmul_bias(x, w, b, block_rows):
    r, din = x.shape
    dout = w.shape[1]
    return pl.pallas_call(
        _matmul_bias_body,
        grid=(r // block_rows,),
        in_specs=[
            pl.BlockSpec((block_rows, din), lambda i: (i, 0)),
            pl.BlockSpec((din, dout), lambda i: (0, 0)),
            pl.BlockSpec((1, dout), lambda i: (0, 0)),
        ],
        out_specs=pl.BlockSpec((block_rows, dout), lambda i: (i, 0)),
        out_shape=jax.ShapeDtypeStruct((r, dout), F32),
    )(x, w, b.reshape(1, -1))


def _enc_tables_body(x_ref, oh_ref, w1_ref, b1_ref, w2_ref, b2_ref,
                     a_ref, bm_ref, ud_ref, b1e_ref,
                     xh_ref, ga_ref, gb_ref):
    hid = jnp.dot(x_ref[...], w1_ref[...], preferred_element_type=F32) + b1_ref[...]
    hid = jnp.maximum(hid, 0.0)
    xh = jnp.dot(hid, w2_ref[...], preferred_element_type=F32) + b2_ref[...]
    xh_ref[...] = xh
    ga = jnp.dot(xh, a_ref[...], preferred_element_type=F32)
    ga += jnp.dot(oh_ref[...], ud_ref[...], preferred_element_type=F32)
    ga_ref[...] = ga + b1e_ref[...]
    gb_ref[...] = jnp.dot(xh, bm_ref[...], preferred_element_type=F32)


def _enc_tables(x, oh, w1, b1, w2, b2, a, bm, ud, b1e):
    r, din = x.shape
    nb = oh.shape[1]
    h = w2.shape[1]
    full = lambda d0, d1: pl.BlockSpec((d0, d1), lambda i: (0, 0))
    rows = lambda d1: pl.BlockSpec((NODE_BLK, d1), lambda i: (i, 0))
    return pl.pallas_call(
        _enc_tables_body,
        grid=(r // NODE_BLK,),
        in_specs=[rows(din), rows(nb), full(din, h), full(1, h), full(h, h),
                  full(1, h), full(h, h), full(h, h), full(nb, h), full(1, h)],
        out_specs=[rows(h), rows(h), rows(h)],
        out_shape=[jax.ShapeDtypeStruct((r, h), F32)] * 3,
    )(x, oh, w1, b1.reshape(1, -1), w2, b2.reshape(1, -1),
      a, bm, ud, b1e.reshape(1, -1))


def _node_update_body(nxt, xh_ref, s0_ref, s1_ref, c0_ref, c1_ref, oh_ref,
                      w2e_ref, b2e_ref, p_ref, q_ref, ur_ref, b1n_ref,
                      w2n_ref, b2n_ref, *rest):
    xh = xh_ref[...]
    s = s0_ref[...] + s1_ref[...]
    cnt = c0_ref[...] + c1_ref[...]
    sums = jnp.dot(s, w2e_ref[...], preferred_element_type=F32) + cnt * b2e_ref[...]
    agg = sums / jnp.maximum(cnt, 1.0)
    pre = jnp.dot(xh, p_ref[...], preferred_element_type=F32)
    pre += jnp.dot(agg, q_ref[...], preferred_element_type=F32)
    pre += jnp.dot(oh_ref[...], ur_ref[...], preferred_element_type=F32)
    hid = jnp.maximum(pre + b1n_ref[...], 0.0)
    xh2 = jnp.dot(hid, w2n_ref[...], preferred_element_type=F32) + b2n_ref[...] + xh
    if nxt == 'tables':
        a_ref, bm_ref, ud_ref, b1e_ref, o_ref, ga_ref, gb_ref = rest
        o_ref[...] = xh2
        ga = jnp.dot(xh2, a_ref[...], preferred_element_type=F32)
        ga += jnp.dot(oh_ref[...], ud_ref[...], preferred_element_type=F32)
        ga_ref[...] = ga + b1e_ref[...]
        gb_ref[...] = jnp.dot(xh2, bm_ref[...], preferred_element_type=F32)
    else:
        w1d_ref, b1d_ref, w2d_ref, b2d_ref, o_ref = rest
        hd = jnp.dot(xh2, w1d_ref[...], preferred_element_type=F32) + b1d_ref[...]
        hd = jnp.maximum(hd, 0.0)
        o_ref[...] = jnp.dot(hd, w2d_ref[...], preferred_element_type=F32) + b2d_ref[...]


def _node_update(xh, s0, s1, c0, c1, oh, w2e, b2e, p, q, ur, b1n, w2n, b2n,
                 nxt, extra):
    r, h = xh.shape
    nb = oh.shape[1]
    full = lambda d0, d1: pl.BlockSpec((d0, d1), lambda i: (0, 0))
    rows = lambda d1: pl.BlockSpec((NODE_BLK, d1), lambda i: (i, 0))
    in_specs = [rows(h), rows(h), rows(h), rows(1), rows(1), rows(nb),
                full(h, h), full(1, h), full(h, h), full(h, h), full(nb, h),
                full(1, h), full(h, h), full(1, h)]
    args = [xh, s0, s1, c0.reshape(-1, 1), c1.reshape(-1, 1), oh,
            w2e, b2e.reshape(1, -1), p, q, ur, b1n.reshape(1, -1),
            w2n, b2n.reshape(1, -1)]
    if nxt == 'tables':
        a, bm, ud, b1e = extra
        in_specs += [full(h, h), full(h, h), full(nb, h), full(1, h)]
        args += [a, bm, ud, b1e.reshape(1, -1)]
        out_specs = [rows(h), rows(h), rows(h)]
        out_shape = [jax.ShapeDtypeStruct((r, h), F32)] * 3
    else:
        w1d, b1d, w2d, b2d = extra
        dh = w1d.shape[1]
        dout = w2d.shape[1]
        in_specs += [full(h, dh), full(1, dh), full(dh, dout), full(1, dout)]
        args += [w1d, b1d.reshape(1, -1), w2d, b2d.reshape(1, -1)]
        out_specs = rows(dout)
        out_shape = jax.ShapeDtypeStruct((r, dout), F32)
    return pl.pallas_call(
        functools.partial(_node_update_body, nxt),
        grid=(r // NODE_BLK,),
        in_specs=in_specs,
        out_specs=out_specs,
        out_shape=out_shape,
    )(*args)


def _sc_edge_pass(row, col, ehc, ga, gb, zrow, *, first):
    e, h = ehc.shape
    nbkt = NP // h
    try:
        info = plsc.get_sparse_core_info()
        nc, ns = info.num_cores, info.num_subcores
    except Exception:
        nc, ns = 2, 16
    nw = nc * ns
    e_per_w = e // nw
    n_chunks = e_per_w // SC_K
    rows_per_s = NP // ns
    bkt_per_s = 8
    n_bkt_s = nbkt // bkt_per_s
    mesh = plsc.VectorSubcoreMesh(core_axis_name="c", subcore_axis_name="s",
                                  num_cores=nc, num_subcores=ns)

    out_type = []
    if first:
        out_type.append(jax.ShapeDtypeStruct((e, h), F32))
        out_type.append(jax.ShapeDtypeStruct((nc, nbkt, h), F32))
    out_type.append(jax.ShapeDtypeStruct((nc, NP, h), F32))

    scratch = [
        pltpu.VMEM((SC_K,), jnp.int32),
        pltpu.VMEM((SC_K,), jnp.int32),
        pltpu.VMEM((SC_K, h), F32),
        pltpu.VMEM((SC_K, h), F32),
        pltpu.VMEM((SC_K, h), F32),
        pltpu.VMEM_SHARED((NP, h), F32),
    ]
    if first:
        scratch.append(pltpu.VMEM((SC_K,), jnp.int32))
        scratch.append(pltpu.VMEM((SC_K, h), F32))
        scratch.append(pltpu.VMEM_SHARED((nbkt, h), F32))
    scratch += [pltpu.SemaphoreType.DMA] * 6

    @functools.partial(pl.kernel, mesh=mesh, out_type=tuple(out_type),
                       scratch_types=scratch)
    def k(row_hbm, col_hbm, ehc_hbm, ga_hbm, gb_hbm, z_hbm, *rest):
        rest = list(rest)
        h_hbm = rest.pop(0) if first else None
        cnt_hbm = rest.pop(0) if first else None
        s_hbm = rest.pop(0)
        idxr_v = rest.pop(0)
        idxc_v = rest.pop(0)
        ga_v = rest.pop(0)
        gb_v = rest.pop(0)
        eh_v = rest.pop(0)
        s_sh = rest.pop(0)
        idxb_v = rest.pop(0) if first else None
        oh_v = rest.pop(0) if first else None
        cnt_sh = rest.pop(0) if first else None
        sem_i, sem_i2, sem_e, sem_g, sem_g2, sem_h = [rest.pop(0) for _ in range(6)]

        cid = lax.axis_index("c")
        sid = lax.axis_index("s")
        wid = cid * ns + sid

        nslc = pl.ds(sid * rows_per_s, rows_per_s)
        pltpu.sync_copy(z_hbm.at[nslc], s_sh.at[nslc])
        bslc = pl.ds(jnp.minimum(sid, n_bkt_s - 1) * bkt_per_s, bkt_per_s)
        if first:
            @pl.when(sid < n_bkt_s)
            def _():
                pltpu.sync_copy(z_hbm.at[bslc], cnt_sh.at[bslc])
        plsc.subcore_barrier()

        ebase = wid * e_per_w
        iota16 = lax.iota(jnp.int32, 16)

        def chunk_body(c, carry):
            base = ebase + c * SC_K
            eslc = pl.ds(base, SC_K)
            d_r = pltpu.async_copy(row_hbm.at[eslc], idxr_v, sem_i)
            d_c = pltpu.async_copy(col_hbm.at[eslc], idxc_v, sem_i2)
            d_e = pltpu.async_copy(ehc_hbm.at[eslc], eh_v, sem_e)
            d_r.wait()
            d_c.wait()
            d_ga = pltpu.async_copy(ga_hbm.at[idxr_v], ga_v, sem_g)
            d_gb = pltpu.async_copy(gb_hbm.at[idxc_v], gb_v, sem_g2)
            d_e.wait()
            d_ga.wait()
            d_gb.wait()

            if first:
                def bkt_body(g, carry2):
                    sl = pl.ds(g * 16, 16)
                    idxb_v[sl] = lax.shift_right_logical(idxr_v[sl], 7)
                    return carry2

                lax.fori_loop(0, SC_K // 16, bkt_body, 0)

            def grp_body(g, carry2):
                lanes = (lax.rem(idxr_v[pl.ds(g * 16, 16)], jnp.int32(h))
                         if first else None)
                for ei in range(16):
                    i = g * 16 + ei
                    for j in range(h // 16):
                        sl = pl.ds(j * 16, 16)
                        v = ga_v[i, sl] + gb_v[i, sl] + eh_v[i, sl]
                        eh_v[i, sl] = jnp.maximum(v, 0.0)
                        if first:
                            oh_v[i, sl] = jnp.where(
                                iota16 + (j * 16) == lanes[ei],
                                1.0, 0.0).astype(F32)
                return carry2

            lax.fori_loop(0, SC_K // 16, grp_body, 0)
            if first:
                d_h = pltpu.async_copy(eh_v, h_hbm.at[eslc], sem_h)
                d_oh = pltpu.async_copy(oh_v, cnt_sh.at[idxb_v], sem_i,
                                        add=True)
            d_s = pltpu.async_copy(eh_v, s_sh.at[idxr_v], sem_i2, add=True)
            if first:
                d_oh.wait()
                d_h.wait()
            d_s.wait()
            return carry

        lax.fori_loop(0, n_chunks, chunk_body, 0)
        plsc.subcore_barrier()
        pltpu.sync_copy(s_sh.at[nslc], s_hbm.at[cid, nslc])
        if first:
            @pl.when(sid < n_bkt_s)
            def _():
                pltpu.sync_copy(cnt_sh.at[bslc], cnt_hbm.at[cid, bslc])

    return k(row, col, ehc, ga, gb, zrow)


def kernel(x, edge_index, edge_attr, conditions, batch, params):
    n = x.shape[0]
    h = params['node_enc']['w2'].shape[1]
    nb = conditions.shape[0]

    row = edge_index[0].astype(jnp.int32)
    col = edge_index[1].astype(jnp.int32)
    batch = batch.astype(jnp.int32)

    xp = jnp.zeros((NP, x.shape[1]), F32).at[:n].set(x)
    oh = jnp.zeros((NP, nb), F32).at[:n].set(
        (batch[:, None] == jnp.arange(nb, dtype=jnp.int32)[None, :]).astype(F32))
    zrow = jnp.zeros((NP, h), F32)

    l0 = params['layers'][0]
    a0, b0m, c0m, d0m = jnp.split(l0['edge']['w1'], 4, axis=0)
    pc = params['cond_enc']
    u = _mlp2(conditions, pc['w1'], pc['b1'], pc['w2'], pc['b2'], nb)
    pn = params['node_enc']
    xh, ga, gb = _enc_tables(xp, oh, pn['w1'], pn['b1'], pn['w2'], pn['b2'],
                             a0, b0m, u @ d0m, l0['edge']['b1'])

    pe = params['edge_enc']
    ehc = _mlp2(edge_attr, pe['w1'], pe['b1'], pe['w2'] @ c0m, pe['b2'] @ c0m,
                EDGE_BLK)

    l1 = params['layers'][1]
    a1, b1m, c1m, d1m = jnp.split(l1['edge']['w1'], 4, axis=0)
    pd = params['decoder']

    for li in range(2):
        lp = params['layers'][li]
        first = li == 0
        if not first:
            prev = params['layers'][0]['edge']
            ehc = _matmul_bias(h_prev, prev['w2'] @ c1m, prev['b2'] @ c1m,
                               EDGE_BLK)
        outs = _sc_edge_pass(row, col, ehc, ga, gb, zrow, first=first)
        if first:
            h_prev, cnt_p, s_p = outs
            cnt0 = cnt_p[0].reshape(NP)
            cnt1 = cnt_p[1].reshape(NP)
        else:
            (s_p,) = outs
        pm, qm, rm = jnp.split(lp['node']['w1'], 3, axis=0)
        if first:
            xh, ga, gb = _node_update(
                xh, s_p[0], s_p[1], cnt0, cnt1, oh,
                lp['edge']['w2'], lp['edge']['b2'],
                pm, qm, u @ rm, lp['node']['b1'],
                lp['node']['w2'], lp['node']['b2'],
                'tables', (a1, b1m, u @ d1m, l1['edge']['b1']))
        else:
            out = _node_update(
                xh, s_p[0], s_p[1], cnt0, cnt1, oh,
                lp['edge']['w2'], lp['edge']['b2'],
                pm, qm, u @ rm, lp['node']['b1'],
                lp['node']['w2'], lp['node']['b2'],
                'decoder', (pd['w1'], pd['b1'], pd['w2'], pd['b2']))

    return out[:n]


# speedup vs baseline: 4.4094x; 1.0886x over previous
import functools

import jax
import jax.numpy as jnp
from jax import lax
from jax.experimental import pallas as pl
from jax.experimental.pallas import tpu as pltpu
from jax.experimental.pallas import tpu_sc as plsc

F32 = jnp.float32
NP = 10240
NODE_BLK = 2048
EDGE_BLK = 4000
SC_K = 80


def _mlp2_body(x_ref, w1_ref, b1_ref, w2_ref, b2_ref, o_ref):
    h = jnp.dot(x_ref[...], w1_ref[...], preferred_element_type=F32) + b1_ref[...]
    h = jnp.maximum(h, 0.0)
    o_ref[...] = jnp.dot(h, w2_ref[...], preferred_element_type=F32) + b2_ref[...]


def _mlp2(x, w1, b1, w2, b2, block_rows):
    r, din = x.shape
    dh = w1.shape[1]
    dout = w2.shape[1]
    return pl.pallas_call(
        _mlp2_body,
        grid=(r // block_rows,),
        in_specs=[
            pl.BlockSpec((block_rows, din), lambda i: (i, 0)),
            pl.BlockSpec((din, dh), lambda i: (0, 0)),
            pl.BlockSpec((1, dh), lambda i: (0, 0)),
            pl.BlockSpec((dh, dout), lambda i: (0, 0)),
            pl.BlockSpec((1, dout), lambda i: (0, 0)),
        ],
        out_specs=pl.BlockSpec((block_rows, dout), lambda i: (i, 0)),
        out_shape=jax.ShapeDtypeStruct((r, dout), F32),
    )(x, w1, b1.reshape(1, -1), w2, b2.reshape(1, -1))


def _matmul_bias_body(x_ref, w_ref, b_ref, o_ref):
    o_ref[...] = jnp.dot(x_ref[...], w_ref[...], preferred_element_type=F32) + b_ref[...]


def _matmul_bias(x, w, b, block_rows):
    r, din = x.shape
    dout = w.shape[1]
    return pl.pallas_call(
        _matmul_bias_body,
        grid=(r // block_rows,),
        in_specs=[
            pl.BlockSpec((block_rows, din), lambda i: (i, 0)),
            pl.BlockSpec((din, dout), lambda i: (0, 0)),
            pl.BlockSpec((1, dout), lambda i: (0, 0)),
        ],
        out_specs=pl.BlockSpec((block_rows, dout), lambda i: (i, 0)),
        out_shape=jax.ShapeDtypeStruct((r, dout), F32),
    )(x, w, b.reshape(1, -1))


def _enc_tables_body(x_ref, oh_ref, w1_ref, b1_ref, w2_ref, b2_ref,
                     a_ref, bm_ref, ud_ref, b1e_ref,
                     xh_ref, ga_ref, gb_ref):
    hid = jnp.dot(x_ref[...], w1_ref[...], preferred_element_type=F32) + b1_ref[...]
    hid = jnp.maximum(hid, 0.0)
    xh = jnp.dot(hid, w2_ref[...], preferred_element_type=F32) + b2_ref[...]
    xh_ref[...] = xh
    ga = jnp.dot(xh, a_ref[...], preferred_element_type=F32)
    ga += jnp.dot(oh_ref[...], ud_ref[...], preferred_element_type=F32)
    ga_ref[...] = ga + b1e_ref[...]
    gb_ref[...] = jnp.dot(xh, bm_ref[...], preferred_element_type=F32)


def _enc_tables(x, oh, w1, b1, w2, b2, a, bm, ud, b1e):
    r, din = x.shape
    nb = oh.shape[1]
    h = w2.shape[1]
    full = lambda d0, d1: pl.BlockSpec((d0, d1), lambda i: (0, 0))
    rows = lambda d1: pl.BlockSpec((NODE_BLK, d1), lambda i: (i, 0))
    return pl.pallas_call(
        _enc_tables_body,
        grid=(r // NODE_BLK,),
        in_specs=[rows(din), rows(nb), full(din, h), full(1, h), full(h, h),
                  full(1, h), full(h, h), full(h, h), full(nb, h), full(1, h)],
        out_specs=[rows(h), rows(h), rows(h)],
        out_shape=[jax.ShapeDtypeStruct((r, h), F32)] * 3,
    )(x, oh, w1, b1.reshape(1, -1), w2, b2.reshape(1, -1),
      a, bm, ud, b1e.reshape(1, -1))


def _node_update_body(nxt, xh_ref, s0_ref, s1_ref, c0_ref, c1_ref, oh_ref,
                      w2e_ref, b2e_ref, p_ref, q_ref, ur_ref, b1n_ref,
                      w2n_ref, b2n_ref, *rest):
    xh = xh_ref[...]
    s = s0_ref[...] + s1_ref[...]
    cnt = c0_ref[...] + c1_ref[...]
    sums = jnp.dot(s, w2e_ref[...], preferred_element_type=F32) + cnt * b2e_ref[...]
    agg = sums / jnp.maximum(cnt, 1.0)
    pre = jnp.dot(xh, p_ref[...], preferred_element_type=F32)
    pre += jnp.dot(agg, q_ref[...], preferred_element_type=F32)
    pre += jnp.dot(oh_ref[...], ur_ref[...], preferred_element_type=F32)
    hid = jnp.maximum(pre + b1n_ref[...], 0.0)
    xh2 = jnp.dot(hid, w2n_ref[...], preferred_element_type=F32) + b2n_ref[...] + xh
    if nxt == 'tables':
        a_ref, bm_ref, ud_ref, b1e_ref, o_ref, ga_ref, gb_ref = rest
        o_ref[...] = xh2
        ga = jnp.dot(xh2, a_ref[...], preferred_element_type=F32)
        ga += jnp.dot(oh_ref[...], ud_ref[...], preferred_element_type=F32)
        ga_ref[...] = ga + b1e_ref[...]
        gb_ref[...] = jnp.dot(xh2, bm_ref[...], preferred_element_type=F32)
    else:
        w1d_ref, b1d_ref, w2d_ref, b2d_ref, o_ref = rest
        hd = jnp.dot(xh2, w1d_ref[...], preferred_element_type=F32) + b1d_ref[...]
        hd = jnp.maximum(hd, 0.0)
        o_ref[...] = jnp.dot(hd, w2d_ref[...], preferred_element_type=F32) + b2d_ref[...]


def _node_update(xh, s0, s1, c0, c1, oh, w2e, b2e, p, q, ur, b1n, w2n, b2n,
                 nxt, extra):
    r, h = xh.shape
    nb = oh.shape[1]
    full = lambda d0, d1: pl.BlockSpec((d0, d1), lambda i: (0, 0))
    rows = lambda d1: pl.BlockSpec((NODE_BLK, d1), lambda i: (i, 0))
    in_specs = [rows(h), rows(h), rows(h), rows(1), rows(1), rows(nb),
                full(h, h), full(1, h), full(h, h), full(h, h), full(nb, h),
                full(1, h), full(h, h), full(1, h)]
    args = [xh, s0, s1, c0.reshape(-1, 1), c1.reshape(-1, 1), oh,
            w2e, b2e.reshape(1, -1), p, q, ur, b1n.reshape(1, -1),
            w2n, b2n.reshape(1, -1)]
    if nxt == 'tables':
        a, bm, ud, b1e = extra
        in_specs += [full(h, h), full(h, h), full(nb, h), full(1, h)]
        args += [a, bm, ud, b1e.reshape(1, -1)]
        out_specs = [rows(h), rows(h), rows(h)]
        out_shape = [jax.ShapeDtypeStruct((r, h), F32)] * 3
    else:
        w1d, b1d, w2d, b2d = extra
        dh = w1d.shape[1]
        dout = w2d.shape[1]
        in_specs += [full(h, dh), full(1, dh), full(dh, dout), full(1, dout)]
        args += [w1d, b1d.reshape(1, -1), w2d, b2d.reshape(1, -1)]
        out_specs = rows(dout)
        out_shape = jax.ShapeDtypeStruct((r, dout), F32)
    return pl.pallas_call(
        functools.partial(_node_update_body, nxt),
        grid=(r // NODE_BLK,),
        in_specs=in_specs,
        out_specs=out_specs,
        out_shape=out_shape,
    )(*args)


def _sc_edge_pass(row, col, ehc, ga, gb, zrow, *, first):
    e, h = ehc.shape
    nbkt = NP // h
    try:
        info = plsc.get_sparse_core_info()
        nc, ns = info.num_cores, info.num_subcores
    except Exception:
        nc, ns = 2, 16
    nw = nc * ns
    e_per_w = e // nw
    n_chunks = e_per_w // SC_K
    rows_per_s = NP // ns
    bkt_per_s = 8
    n_bkt_s = nbkt // bkt_per_s
    mesh = plsc.VectorSubcoreMesh(core_axis_name="c", subcore_axis_name="s",
                                  num_cores=nc, num_subcores=ns)

    out_type = []
    if first:
        out_type.append(jax.ShapeDtypeStruct((e, h), F32))
        out_type.append(jax.ShapeDtypeStruct((nc, nbkt, h), F32))
    out_type.append(jax.ShapeDtypeStruct((nc, NP, h), F32))

    scratch = [
        pltpu.VMEM((SC_K,), jnp.int32),
        pltpu.VMEM((SC_K,), jnp.int32),
        pltpu.VMEM((SC_K, h), F32),
        pltpu.VMEM((SC_K, h), F32),
        pltpu.VMEM((SC_K, h), F32),
        pltpu.VMEM_SHARED((NP, h), F32),
    ]
    if first:
        scratch.append(pltpu.VMEM((SC_K,), jnp.int32))
        scratch.append(pltpu.VMEM((SC_K, h), F32))
        scratch.append(pltpu.VMEM_SHARED((nbkt, h), F32))
    scratch += [pltpu.SemaphoreType.DMA] * 6

    @functools.partial(pl.kernel, mesh=mesh, out_type=tuple(out_type),
                       scratch_types=scratch)
    def k(row_hbm, col_hbm, ehc_hbm, ga_hbm, gb_hbm, z_hbm, *rest):
        rest = list(rest)
        h_hbm = rest.pop(0) if first else None
        cnt_hbm = rest.pop(0) if first else None
        s_hbm = rest.pop(0)
        idxr_v = rest.pop(0)
        idxc_v = rest.pop(0)
        ga_v = rest.pop(0)
        gb_v = rest.pop(0)
        eh_v = rest.pop(0)
        s_sh = rest.pop(0)
        idxb_v = rest.pop(0) if first else None
        oh_v = rest.pop(0) if first else None
        cnt_sh = rest.pop(0) if first else None
        sem_i, sem_i2, sem_e, sem_g, sem_g2, sem_h = [rest.pop(0) for _ in range(6)]

        cid = lax.axis_index("c")
        sid = lax.axis_index("s")
        wid = cid * ns + sid

        nslc = pl.ds(sid * rows_per_s, rows_per_s)
        pltpu.sync_copy(z_hbm.at[nslc], s_sh.at[nslc])
        bslc = pl.ds(jnp.minimum(sid, n_bkt_s - 1) * bkt_per_s, bkt_per_s)
        if first:
            @pl.when(sid < n_bkt_s)
            def _():
                pltpu.sync_copy(z_hbm.at[bslc], cnt_sh.at[bslc])
        plsc.subcore_barrier()

        ebase = wid * e_per_w
        iota16 = lax.iota(jnp.int32, 16)

        def chunk_body(c, carry):
            base = ebase + c * SC_K
            eslc = pl.ds(base, SC_K)
            d_r = pltpu.async_copy(row_hbm.at[eslc], idxr_v, sem_i)
            d_c = pltpu.async_copy(col_hbm.at[eslc], idxc_v, sem_i2)
            d_e = pltpu.async_copy(ehc_hbm.at[eslc], eh_v, sem_e)
            d_r.wait()
            d_c.wait()
            d_ga = pltpu.async_copy(ga_hbm.at[idxr_v], ga_v, sem_g)
            d_gb = pltpu.async_copy(gb_hbm.at[idxc_v], gb_v, sem_g2)
            d_e.wait()
            d_ga.wait()
            d_gb.wait()

            if first:
                def bkt_body(g, carry2):
                    sl = pl.ds(g * 16, 16)
                    idxb_v[sl] = lax.shift_right_logical(idxr_v[sl], 7)
                    return carry2

                lax.fori_loop(0, SC_K // 16, bkt_body, 0)

            def grp_body(g, carry2):
                lanes = (lax.rem(idxr_v[pl.ds(g * 16, 16)], jnp.int32(h))
                         if first else None)
                for ei in range(16):
                    i = g * 16 + ei
                    for j in range(h // 16):
                        sl = pl.ds(j * 16, 16)
                        v = ga_v[i, sl] + gb_v[i, sl] + eh_v[i, sl]
                        eh_v[i, sl] = jnp.maximum(v, 0.0)
                        if first:
                            oh_v[i, sl] = jnp.where(
                                iota16 + (j * 16) == lanes[ei],
                                1.0, 0.0).astype(F32)
                return carry2

            lax.fori_loop(0, SC_K // 16, grp_body, 0)
            if first:
                d_h = pltpu.async_copy(eh_v, h_hbm.at[eslc], sem_h)
                d_oh = pltpu.async_copy(oh_v, cnt_sh.at[idxb_v], sem_i,
                                        add=True)
            d_s = pltpu.async_copy(eh_v, s_sh.at[idxr_v], sem_i2, add=True)
            if first:
                d_oh.wait()
                d_h.wait()
            d_s.wait()
            return carry

        lax.fori_loop(0, n_chunks, chunk_body, 0)
        plsc.subcore_barrier()
        pltpu.sync_copy(s_sh.at[nslc], s_hbm.at[cid, nslc])
        if first:
            @pl.when(sid < n_bkt_s)
            def _():
                pltpu.sync_copy(cnt_sh.at[bslc], cnt_hbm.at[cid, bslc])

    return k(row, col, ehc, ga, gb, zrow)


def kernel(x, edge_index, edge_attr, conditions, batch, params):
    n = x.shape[0]
    h = params['node_enc']['w2'].shape[1]
    nb = conditions.shape[0]

    row = edge_index[0].astype(jnp.int32)
    col = edge_index[1].astype(jnp.int32)
    batch = batch.astype(jnp.int32)

    xp = jnp.zeros((NP, x.shape[1]), F32).at[:n].set(x)
    oh = jnp.zeros((NP, nb), F32).at[:n].set(
        (batch[:, None] == jnp.arange(nb, dtype=jnp.int32)[None, :]).astype(F32))
    zrow = jnp.zeros((NP, h), F32)

    l0 = params['layers'][0]
    a0, b0m, c0m, d0m = jnp.split(l0['edge']['w1'], 4, axis=0)
    pc = params['cond_enc']
    u = _mlp2(conditions, pc['w1'], pc['b1'], pc['w2'], pc['b2'], nb)
    pn = params['node_enc']
    xh, ga, gb = _enc_tables(xp, oh, pn['w1'], pn['b1'], pn['w2'], pn['b2'],
                             a0, b0m, u @ d0m, l0['edge']['b1'])

    pe = params['edge_enc']
    ehc = _mlp2(edge_attr, pe['w1'], pe['b1'], pe['w2'] @ c0m, pe['b2'] @ c0m,
                EDGE_BLK)

    l1 = params['layers'][1]
    a1, b1m, c1m, d1m = jnp.split(l1['edge']['w1'], 4, axis=0)
    pd = params['decoder']

    for li in range(2):
        lp = params['layers'][li]
        first = li == 0
        if not first:
            prev = params['layers'][0]['edge']
            ehc = _matmul_bias(h_prev, prev['w2'] @ c1m, prev['b2'] @ c1m,
                               EDGE_BLK)
        outs = _sc_edge_pass(row, col, ehc, ga, gb, zrow, first=first)
        if first:
            h_prev, cnt_p, s_p = outs
            cnt0 = cnt_p[0].reshape(NP)
            cnt1 = cnt_p[1].reshape(NP)
        else:
            (s_p,) = outs
        pm, qm, rm = jnp.split(lp['node']['w1'], 3, axis=0)
        if first:
            xh, ga, gb = _node_update(
                xh, s_p[0], s_p[1], cnt0, cnt1, oh,
                lp['edge']['w2'], lp['edge']['b2'],
                pm, qm, u @ rm, lp['node']['b1'],
                lp['node']['w2'], lp['node']['b2'],
                'tables', (a1, b1m, u @ d1m, l1['edge']['b1']))
        else:
            out = _node_update(
                xh, s_p[0], s_p[1], cnt0, cnt1, oh,
                lp['edge']['w2'], lp['edge']['b2'],
                pm, qm, u @ rm, lp['node']['b1'],
                lp['node']['w2'], lp['node']['b2'],
                'decoder', (pd['w1'], pd['b1'], pd['w2'], pd['b2']))

    return out[:n]


# speedup vs baseline: 4.5172x; 1.0245x over previous
import functools

import jax
import jax.numpy as jnp
from jax import lax
from jax.experimental import pallas as pl
from jax.experimental.pallas import tpu as pltpu
from jax.experimental.pallas import tpu_sc as plsc

F32 = jnp.float32
NP = 10240
NODE_BLK = 5120
EDGE_BLK = 8000
SC_K = 80


def _mlp2_body(x_ref, w1_ref, b1_ref, w2_ref, b2_ref, o_ref):
    h = jnp.dot(x_ref[...], w1_ref[...], preferred_element_type=F32) + b1_ref[...]
    h = jnp.maximum(h, 0.0)
    o_ref[...] = jnp.dot(h, w2_ref[...], preferred_element_type=F32) + b2_ref[...]


def _mlp2(x, w1, b1, w2, b2, block_rows):
    r, din = x.shape
    dh = w1.shape[1]
    dout = w2.shape[1]
    return pl.pallas_call(
        _mlp2_body,
        grid=(r // block_rows,),
        in_specs=[
            pl.BlockSpec((block_rows, din), lambda i: (i, 0)),
            pl.BlockSpec((din, dh), lambda i: (0, 0)),
            pl.BlockSpec((1, dh), lambda i: (0, 0)),
            pl.BlockSpec((dh, dout), lambda i: (0, 0)),
            pl.BlockSpec((1, dout), lambda i: (0, 0)),
        ],
        out_specs=pl.BlockSpec((block_rows, dout), lambda i: (i, 0)),
        out_shape=jax.ShapeDtypeStruct((r, dout), F32),
    )(x, w1, b1.reshape(1, -1), w2, b2.reshape(1, -1))


def _matmul_bias_body(x_ref, w_ref, b_ref, o_ref):
    o_ref[...] = jnp.dot(x_ref[...], w_ref[...], preferred_element_type=F32) + b_ref[...]


def _matmul_bias(x, w, b, block_rows):
    r, din = x.shape
    dout = w.shape[1]
    return pl.pallas_call(
        _matmul_bias_body,
        grid=(r // block_rows,),
        in_specs=[
            pl.BlockSpec((block_rows, din), lambda i: (i, 0)),
            pl.BlockSpec((din, dout), lambda i: (0, 0)),
            pl.BlockSpec((1, dout), lambda i: (0, 0)),
        ],
        out_specs=pl.BlockSpec((block_rows, dout), lambda i: (i, 0)),
        out_shape=jax.ShapeDtypeStruct((r, dout), F32),
    )(x, w, b.reshape(1, -1))


def _enc_tables_body(x_ref, oh_ref, w1_ref, b1_ref, w2_ref, b2_ref,
                     a_ref, bm_ref, ud_ref, b1e_ref,
                     xh_ref, ga_ref, gb_ref):
    hid = jnp.dot(x_ref[...], w1_ref[...], preferred_element_type=F32) + b1_ref[...]
    hid = jnp.maximum(hid, 0.0)
    xh = jnp.dot(hid, w2_ref[...], preferred_element_type=F32) + b2_ref[...]
    xh_ref[...] = xh
    ga = jnp.dot(xh, a_ref[...], preferred_element_type=F32)
    ga += jnp.dot(oh_ref[...], ud_ref[...], preferred_element_type=F32)
    ga_ref[...] = ga + b1e_ref[...]
    gb_ref[...] = jnp.dot(xh, bm_ref[...], preferred_element_type=F32)


def _enc_tables(x, oh, w1, b1, w2, b2, a, bm, ud, b1e):
    r, din = x.shape
    nb = oh.shape[1]
    h = w2.shape[1]
    full = lambda d0, d1: pl.BlockSpec((d0, d1), lambda i: (0, 0))
    rows = lambda d1: pl.BlockSpec((NODE_BLK, d1), lambda i: (i, 0))
    return pl.pallas_call(
        _enc_tables_body,
        grid=(r // NODE_BLK,),
        in_specs=[rows(din), rows(nb), full(din, h), full(1, h), full(h, h),
                  full(1, h), full(h, h), full(h, h), full(nb, h), full(1, h)],
        out_specs=[rows(h), rows(h), rows(h)],
        out_shape=[jax.ShapeDtypeStruct((r, h), F32)] * 3,
    )(x, oh, w1, b1.reshape(1, -1), w2, b2.reshape(1, -1),
      a, bm, ud, b1e.reshape(1, -1))


def _node_update_body(nxt, xh_ref, s0_ref, s1_ref, c0_ref, c1_ref, oh_ref,
                      w2e_ref, b2e_ref, p_ref, q_ref, ur_ref, b1n_ref,
                      w2n_ref, b2n_ref, *rest):
    xh = xh_ref[...]
    s = s0_ref[...] + s1_ref[...]
    cnt = c0_ref[...] + c1_ref[...]
    sums = jnp.dot(s, w2e_ref[...], preferred_element_type=F32) + cnt * b2e_ref[...]
    agg = sums / jnp.maximum(cnt, 1.0)
    pre = jnp.dot(xh, p_ref[...], preferred_element_type=F32)
    pre += jnp.dot(agg, q_ref[...], preferred_element_type=F32)
    pre += jnp.dot(oh_ref[...], ur_ref[...], preferred_element_type=F32)
    hid = jnp.maximum(pre + b1n_ref[...], 0.0)
    xh2 = jnp.dot(hid, w2n_ref[...], preferred_element_type=F32) + b2n_ref[...] + xh
    if nxt == 'tables':
        a_ref, bm_ref, ud_ref, b1e_ref, o_ref, ga_ref, gb_ref = rest
        o_ref[...] = xh2
        ga = jnp.dot(xh2, a_ref[...], preferred_element_type=F32)
        ga += jnp.dot(oh_ref[...], ud_ref[...], preferred_element_type=F32)
        ga_ref[...] = ga + b1e_ref[...]
        gb_ref[...] = jnp.dot(xh2, bm_ref[...], preferred_element_type=F32)
    else:
        w1d_ref, b1d_ref, w2d_ref, b2d_ref, o_ref = rest
        hd = jnp.dot(xh2, w1d_ref[...], preferred_element_type=F32) + b1d_ref[...]
        hd = jnp.maximum(hd, 0.0)
        o_ref[...] = jnp.dot(hd, w2d_ref[...], preferred_element_type=F32) + b2d_ref[...]


def _node_update(xh, s0, s1, c0, c1, oh, w2e, b2e, p, q, ur, b1n, w2n, b2n,
                 nxt, extra):
    r, h = xh.shape
    nb = oh.shape[1]
    full = lambda d0, d1: pl.BlockSpec((d0, d1), lambda i: (0, 0))
    rows = lambda d1: pl.BlockSpec((NODE_BLK, d1), lambda i: (i, 0))
    in_specs = [rows(h), rows(h), rows(h), rows(1), rows(1), rows(nb),
                full(h, h), full(1, h), full(h, h), full(h, h), full(nb, h),
                full(1, h), full(h, h), full(1, h)]
    args = [xh, s0, s1, c0.reshape(-1, 1), c1.reshape(-1, 1), oh,
            w2e, b2e.reshape(1, -1), p, q, ur, b1n.reshape(1, -1),
            w2n, b2n.reshape(1, -1)]
    if nxt == 'tables':
        a, bm, ud, b1e = extra
        in_specs += [full(h, h), full(h, h), full(nb, h), full(1, h)]
        args += [a, bm, ud, b1e.reshape(1, -1)]
        out_specs = [rows(h), rows(h), rows(h)]
        out_shape = [jax.ShapeDtypeStruct((r, h), F32)] * 3
    else:
        w1d, b1d, w2d, b2d = extra
        dh = w1d.shape[1]
        dout = w2d.shape[1]
        in_specs += [full(h, dh), full(1, dh), full(dh, dout), full(1, dout)]
        args += [w1d, b1d.reshape(1, -1), w2d, b2d.reshape(1, -1)]
        out_specs = rows(dout)
        out_shape = jax.ShapeDtypeStruct((r, dout), F32)
    return pl.pallas_call(
        functools.partial(_node_update_body, nxt),
        grid=(r // NODE_BLK,),
        in_specs=in_specs,
        out_specs=out_specs,
        out_shape=out_shape,
    )(*args)


def _sc_edge_pass(row, col, ehc, ga, gb, zrow, *, first):
    e, h = ehc.shape
    nbkt = NP // h
    try:
        info = plsc.get_sparse_core_info()
        nc, ns = info.num_cores, info.num_subcores
    except Exception:
        nc, ns = 2, 16
    nw = nc * ns
    e_per_w = e // nw
    n_chunks = e_per_w // SC_K
    rows_per_s = NP // ns
    bkt_per_s = 8
    n_bkt_s = nbkt // bkt_per_s
    mesh = plsc.VectorSubcoreMesh(core_axis_name="c", subcore_axis_name="s",
                                  num_cores=nc, num_subcores=ns)

    out_type = []
    if first:
        out_type.append(jax.ShapeDtypeStruct((e, h), F32))
        out_type.append(jax.ShapeDtypeStruct((nc, nbkt, h), F32))
    out_type.append(jax.ShapeDtypeStruct((nc, NP, h), F32))

    scratch = [
        pltpu.VMEM((SC_K,), jnp.int32),
        pltpu.VMEM((SC_K,), jnp.int32),
        pltpu.VMEM((SC_K, h), F32),
        pltpu.VMEM((SC_K, h), F32),
        pltpu.VMEM((SC_K, h), F32),
        pltpu.VMEM_SHARED((NP, h), F32),
    ]
    if first:
        scratch.append(pltpu.VMEM((SC_K,), jnp.int32))
        scratch.append(pltpu.VMEM((SC_K, h), F32))
        scratch.append(pltpu.VMEM_SHARED((nbkt, h), F32))
    scratch += [pltpu.SemaphoreType.DMA] * 6

    @functools.partial(pl.kernel, mesh=mesh, out_type=tuple(out_type),
                       scratch_types=scratch)
    def k(row_hbm, col_hbm, ehc_hbm, ga_hbm, gb_hbm, z_hbm, *rest):
        rest = list(rest)
        h_hbm = rest.pop(0) if first else None
        cnt_hbm = rest.pop(0) if first else None
        s_hbm = rest.pop(0)
        idxr_v = rest.pop(0)
        idxc_v = rest.pop(0)
        ga_v = rest.pop(0)
        gb_v = rest.pop(0)
        eh_v = rest.pop(0)
        s_sh = rest.pop(0)
        idxb_v = rest.pop(0) if first else None
        oh_v = rest.pop(0) if first else None
        cnt_sh = rest.pop(0) if first else None
        sem_i, sem_i2, sem_e, sem_g, sem_g2, sem_h = [rest.pop(0) for _ in range(6)]

        cid = lax.axis_index("c")
        sid = lax.axis_index("s")
        wid = cid * ns + sid

        nslc = pl.ds(sid * rows_per_s, rows_per_s)
        pltpu.sync_copy(z_hbm.at[nslc], s_sh.at[nslc])
        bslc = pl.ds(jnp.minimum(sid, n_bkt_s - 1) * bkt_per_s, bkt_per_s)
        if first:
            @pl.when(sid < n_bkt_s)
            def _():
                pltpu.sync_copy(z_hbm.at[bslc], cnt_sh.at[bslc])
        plsc.subcore_barrier()

        ebase = wid * e_per_w
        iota16 = lax.iota(jnp.int32, 16)

        def chunk_body(c, carry):
            base = ebase + c * SC_K
            eslc = pl.ds(base, SC_K)
            d_r = pltpu.async_copy(row_hbm.at[eslc], idxr_v, sem_i)
            d_c = pltpu.async_copy(col_hbm.at[eslc], idxc_v, sem_i2)
            d_e = pltpu.async_copy(ehc_hbm.at[eslc], eh_v, sem_e)
            d_r.wait()
            d_c.wait()
            d_ga = pltpu.async_copy(ga_hbm.at[idxr_v], ga_v, sem_g)
            d_gb = pltpu.async_copy(gb_hbm.at[idxc_v], gb_v, sem_g2)
            d_e.wait()
            d_ga.wait()
            d_gb.wait()

            if first:
                def bkt_body(g, carry2):
                    sl = pl.ds(g * 16, 16)
                    idxb_v[sl] = lax.shift_right_logical(idxr_v[sl], 7)
                    return carry2

                lax.fori_loop(0, SC_K // 16, bkt_body, 0)

            def grp_body(g, carry2):
                lanes = (lax.rem(idxr_v[pl.ds(g * 16, 16)], jnp.int32(h))
                         if first else None)
                for ei in range(16):
                    i = g * 16 + ei
                    for j in range(h // 16):
                        sl = pl.ds(j * 16, 16)
                        v = ga_v[i, sl] + gb_v[i, sl] + eh_v[i, sl]
                        eh_v[i, sl] = jnp.maximum(v, 0.0)
                        if first:
                            oh_v[i, sl] = jnp.where(
                                iota16 + (j * 16) == lanes[ei],
                                1.0, 0.0).astype(F32)
                return carry2

            lax.fori_loop(0, SC_K // 16, grp_body, 0)
            if first:
                d_h = pltpu.async_copy(eh_v, h_hbm.at[eslc], sem_h)
                d_oh = pltpu.async_copy(oh_v, cnt_sh.at[idxb_v], sem_i,
                                        add=True)
            d_s = pltpu.async_copy(eh_v, s_sh.at[idxr_v], sem_i2, add=True)
            if first:
                d_oh.wait()
                d_h.wait()
            d_s.wait()
            return carry

        lax.fori_loop(0, n_chunks, chunk_body, 0)
        plsc.subcore_barrier()
        pltpu.sync_copy(s_sh.at[nslc], s_hbm.at[cid, nslc])
        if first:
            @pl.when(sid < n_bkt_s)
            def _():
                pltpu.sync_copy(cnt_sh.at[bslc], cnt_hbm.at[cid, bslc])

    return k(row, col, ehc, ga, gb, zrow)


def kernel(x, edge_index, edge_attr, conditions, batch, params):
    n = x.shape[0]
    h = params['node_enc']['w2'].shape[1]
    nb = conditions.shape[0]

    row = edge_index[0].astype(jnp.int32)
    col = edge_index[1].astype(jnp.int32)
    batch = batch.astype(jnp.int32)

    xp = jnp.zeros((NP, x.shape[1]), F32).at[:n].set(x)
    oh = jnp.zeros((NP, nb), F32).at[:n].set(
        (batch[:, None] == jnp.arange(nb, dtype=jnp.int32)[None, :]).astype(F32))
    zrow = jnp.zeros((NP, h), F32)

    l0 = params['layers'][0]
    a0, b0m, c0m, d0m = jnp.split(l0['edge']['w1'], 4, axis=0)
    pc = params['cond_enc']
    u = _mlp2(conditions, pc['w1'], pc['b1'], pc['w2'], pc['b2'], nb)
    pn = params['node_enc']
    xh, ga, gb = _enc_tables(xp, oh, pn['w1'], pn['b1'], pn['w2'], pn['b2'],
                             a0, b0m, u @ d0m, l0['edge']['b1'])

    pe = params['edge_enc']
    ehc = _mlp2(edge_attr, pe['w1'], pe['b1'], pe['w2'] @ c0m, pe['b2'] @ c0m,
                EDGE_BLK)

    l1 = params['layers'][1]
    a1, b1m, c1m, d1m = jnp.split(l1['edge']['w1'], 4, axis=0)
    pd = params['decoder']

    for li in range(2):
        lp = params['layers'][li]
        first = li == 0
        if not first:
            prev = params['layers'][0]['edge']
            ehc = _matmul_bias(h_prev, prev['w2'] @ c1m, prev['b2'] @ c1m,
                               EDGE_BLK)
        outs = _sc_edge_pass(row, col, ehc, ga, gb, zrow, first=first)
        if first:
            h_prev, cnt_p, s_p = outs
            cnt0 = cnt_p[0].reshape(NP)
            cnt1 = cnt_p[1].reshape(NP)
        else:
            (s_p,) = outs
        pm, qm, rm = jnp.split(lp['node']['w1'], 3, axis=0)
        if first:
            xh, ga, gb = _node_update(
                xh, s_p[0], s_p[1], cnt0, cnt1, oh,
                lp['edge']['w2'], lp['edge']['b2'],
                pm, qm, u @ rm, lp['node']['b1'],
                lp['node']['w2'], lp['node']['b2'],
                'tables', (a1, b1m, u @ d1m, l1['edge']['b1']))
        else:
            out = _node_update(
                xh, s_p[0], s_p[1], cnt0, cnt1, oh,
                lp['edge']['w2'], lp['edge']['b2'],
                pm, qm, u @ rm, lp['node']['b1'],
                lp['node']['w2'], lp['node']['b2'],
                'decoder', (pd['w1'], pd['b1'], pd['w2'], pd['b2']))

    return out[:n]
